# baseline (speedup 1.0000x reference)
import functools

import jax
import jax.numpy as jnp
from jax import lax
from jax.experimental import pallas as pl
from jax.experimental.pallas import tpu as pltpu
from jax.experimental.pallas import tpu_sc as plsc

_N = 10000
_E = 320000
_D = 128
_H = 128
_C = 40

_NC = 2
_NS = 16
_NW = _NC * _NS
_EB = 32
_CH = 320
_SEG = 160
_NB = 5
_EPT = _CH * _EB
_E_PAD = _NW * _EPT
_N_PAD = 10240
_RPT = _N_PAD // _NS
_CP = 48

_f32 = jnp.float32
_i32 = jnp.int32

_mesh = plsc.VectorSubcoreMesh(
    core_axis_name="c", subcore_axis_name="s",
    num_cores=_NC, num_subcores=_NS)

_sc_params = pltpu.CompilerParams(needs_layout_passes=False,
                                  use_tc_tiling_on_sc=False)


def _deg_body(col_hbm, ew_hbm, parts_hbm, cidx, ewv, zb, acc):
    c = lax.axis_index("c")
    s = lax.axis_index("s")
    w = c * _NS + s
    for t in range(_RPT // 16):
        zb[pl.ds(t * 16, 16)] = jnp.zeros((16,), _f32)
    pltpu.sync_copy(zb, acc.at[pl.ds(s * _RPT, _RPT)])
    plsc.subcore_barrier()
    pltpu.sync_copy(col_hbm.at[w], cidx)
    pltpu.sync_copy(ew_hbm.at[w], ewv)

    def j_body(j, carry):
        pltpu.sync_copy(ewv.at[j], acc.at[cidx.at[j]], add=True)
        return carry

    lax.fori_loop(0, _CH, j_body, 0)
    plsc.subcore_barrier()
    pltpu.sync_copy(acc.at[pl.ds(s * _RPT, _RPT)],
                    parts_hbm.at[c, pl.ds(s * _RPT, _RPT)])


_deg_call = pl.kernel(
    _deg_body,
    out_type=jax.ShapeDtypeStruct((_NC, _N_PAD), _f32),
    mesh=_mesh,
    compiler_params=_sc_params,
    scratch_types=[
        pltpu.VMEM((_CH, _EB), _i32),
        pltpu.VMEM((_CH, _EB), _f32),
        pltpu.VMEM((_RPT,), _f32),
        pltpu.VMEM_SHARED((_N_PAD,), _f32),
    ],
)


def _norm_body(row_hbm, col_hbm, ew_hbm, dinv_hbm, norm_hbm, dv, rv, cv, ev, nv):
    c = lax.axis_index("c")
    s = lax.axis_index("s")
    w = c * _NS + s
    pltpu.sync_copy(dinv_hbm, dv)
    pltpu.sync_copy(row_hbm.at[w], rv)
    pltpu.sync_copy(col_hbm.at[w], cv)
    pltpu.sync_copy(ew_hbm.at[w], ev)

    def t_body(t, carry):
        r16 = rv[pl.ds(t * 16, 16)]
        c16 = cv[pl.ds(t * 16, 16)]
        e16 = ev[pl.ds(t * 16, 16)]
        n16 = plsc.load_gather(dv, [r16]) * e16 * plsc.load_gather(dv, [c16])
        nv[pl.ds(t * 16, 16)] = n16
        return carry

    lax.fori_loop(0, _EPT // 16, t_body, 0)
    pltpu.sync_copy(nv, norm_hbm.at[w])


_norm_call = pl.kernel(
    _norm_body,
    out_type=jax.ShapeDtypeStruct((_NW, _EPT), _f32),
    mesh=_mesh,
    compiler_params=_sc_params,
    scratch_types=[
        pltpu.VMEM((_N_PAD,), _f32),
        pltpu.VMEM((_EPT,), _i32),
        pltpu.VMEM((_EPT,), _i32),
        pltpu.VMEM((_EPT,), _f32),
        pltpu.VMEM((_EPT,), _f32),
    ],
)


def _agg_body(Wd, z_hbm, row_hbm, col_hbm, norm_hbm, parts_hbm,
              ridx, cidx, nv,
              gb0, gb1, gb2, gb3, gb4, acc,
              gs0, gs1, gs2, gs3, gs4,
              ss0, ss1, ss2, ss3, ss4):
    c = lax.axis_index("c")
    s = lax.axis_index("s")
    w = c * _NS + s
    gbs = (gb0, gb1, gb2, gb3, gb4)
    gss = (gs0, gs1, gs2, gs3, gs4)
    sss = (ss0, ss1, ss2, ss3, ss4)

    def zrow(r, carry):
        for v in range(Wd // 16):
            gb0[r, pl.ds(v * 16, 16)] = jnp.zeros((16,), _f32)
        return carry

    lax.fori_loop(0, _EB, zrow, 0)

    def zcopy(k, carry):
        pltpu.sync_copy(gb0, acc.at[pl.ds(s * _RPT + k * _EB, _EB)])
        return carry

    lax.fori_loop(0, _RPT // _EB, zcopy, 0)
    plsc.subcore_barrier()

    def stage(seg):
        base = seg * _SEG
        pltpu.sync_copy(row_hbm.at[w, pl.ds(base, _SEG)], ridx)
        pltpu.sync_copy(col_hbm.at[w, pl.ds(base, _SEG)], cidx)
        pltpu.sync_copy(norm_hbm.at[w, pl.ds(base, _SEG)], nv)

    def start_gather(l, b):
        pltpu.async_copy(z_hbm.at[ridx.at[l]], gbs[b], gss[b])

    def wait_gather(l, b):
        pltpu.make_async_copy(z_hbm.at[ridx.at[l]], gbs[b], gss[b]).wait()

    def scale(l, b):
        gb = gbs[b]

        @plsc.parallel_loop(0, _EB, 1, unroll=4)
        def _(e):
            nspl = plsc.load_gather(
                nv, [jnp.full((16,), l, _i32), jnp.full((16,), e, _i32)])
            for v in range(Wd // 16):
                sl = pl.ds(v * 16, 16)
                gb[e, sl] = gb[e, sl] * nspl

    def start_scatter(l, b):
        pltpu.async_copy(gbs[b], acc.at[cidx.at[l]], sss[b], add=True)

    def wait_scatter(l, b):
        pltpu.make_async_copy(gbs[b], acc.at[cidx.at[l]], sss[b]).wait()

    def run_seg():
        for b in range(3):
            start_gather(b, b)

        def k_body(k, carry):
            for phase in range(_NB):
                l = k * _NB + phase
                wait_gather(l, phase)
                scale(l, phase)
                start_scatter(l, phase)
                pb = (phase + 3) % _NB

                @pl.when(l >= 2)
                def _():
                    wait_scatter(l - 2, pb)

                @pl.when(l + 3 < _SEG)
                def _():
                    start_gather(l + 3, pb)

            return carry

        lax.fori_loop(0, _SEG // _NB, k_body, 0)
        wait_scatter(_SEG - 2, (_SEG - 2) % _NB)
        wait_scatter(_SEG - 1, (_SEG - 1) % _NB)

    stage(0)
    run_seg()
    stage(1)
    run_seg()
    plsc.subcore_barrier()

    def wb(k, carry):
        pltpu.sync_copy(acc.at[pl.ds(s * _RPT + k * 128, 128)],
                        parts_hbm.at[c, pl.ds(s * _RPT + k * 128, 128)])
        return carry

    lax.fori_loop(0, _RPT // 128, wb, 0)


def _make_agg(Wd):
    return pl.kernel(
        functools.partial(_agg_body, Wd),
        out_type=jax.ShapeDtypeStruct((_NC, _N_PAD, Wd), _f32),
        mesh=_mesh,
        compiler_params=_sc_params,
        scratch_types=(
            [pltpu.VMEM((_SEG, _EB), _i32),
             pltpu.VMEM((_SEG, _EB), _i32),
             pltpu.VMEM((_SEG, _EB), _f32)]
            + [pltpu.VMEM((_EB, Wd), _f32)] * _NB
            + [pltpu.VMEM_SHARED((_N_PAD, Wd), _f32)]
            + [pltpu.SemaphoreType.DMA] * (2 * _NB)
        ),
    )


_agg128 = _make_agg(_H)
_agg48 = _make_agg(_CP)


def _dinv_body(parts_ref, dinv_ref, dinv2_ref):
    deg = 1.0 + parts_ref[0] + parts_ref[1]
    dv = jnp.where(deg > 0, lax.rsqrt(deg), 0.0)
    dinv_ref[...] = dv
    dinv2_ref[...] = dv * dv


def _dinv_call(parts):
    return pl.pallas_call(
        _dinv_body,
        grid=(_N_PAD // 128,),
        in_specs=[pl.BlockSpec((_NC, 128, 1), lambda i: (0, i, 0))],
        out_specs=[pl.BlockSpec((128, 1), lambda i: (i, 0)),
                   pl.BlockSpec((128, 1), lambda i: (i, 0))],
        out_shape=[jax.ShapeDtypeStruct((_N_PAD, 1), _f32),
                   jax.ShapeDtypeStruct((_N_PAD, 1), _f32)],
    )(parts)


def _mm_body(x_ref, w_ref, o_ref):
    o_ref[...] = jnp.dot(x_ref[...], w_ref[...],
                         preferred_element_type=_f32)


def _mm_call(x, w):
    wo = w.shape[1]
    return pl.pallas_call(
        _mm_body,
        grid=(_N_PAD // 128,),
        in_specs=[pl.BlockSpec((128, _D), lambda i: (i, 0)),
                  pl.BlockSpec((_D, wo), lambda i: (0, 0))],
        out_specs=pl.BlockSpec((128, wo), lambda i: (i, 0)),
        out_shape=jax.ShapeDtypeStruct((_N_PAD, wo), _f32),
    )(x, w)


def _epi_body(p0_ref, p1_ref, z_ref, d2_ref, b_ref, w_ref, o_ref):
    h = p0_ref[...] + p1_ref[...] + d2_ref[...] * z_ref[...] + b_ref[...]
    h = jnp.maximum(h, 0.0)
    o_ref[...] = jnp.dot(h, w_ref[...], preferred_element_type=_f32)


def _epi_call(p0, p1, z, d2, b, w):
    wo = w.shape[1]
    return pl.pallas_call(
        _epi_body,
        grid=(_N_PAD // 128,),
        in_specs=[pl.BlockSpec((128, _H), lambda i: (i, 0)),
                  pl.BlockSpec((128, _H), lambda i: (i, 0)),
                  pl.BlockSpec((128, _H), lambda i: (i, 0)),
                  pl.BlockSpec((128, 1), lambda i: (i, 0)),
                  pl.BlockSpec((1, _H), lambda i: (0, 0)),
                  pl.BlockSpec((_H, wo), lambda i: (0, 0))],
        out_specs=pl.BlockSpec((128, wo), lambda i: (i, 0)),
        out_shape=jax.ShapeDtypeStruct((_N_PAD, wo), _f32),
    )(p0, p1, z, d2, b, w)


def _final_body(p0_ref, p1_ref, z_ref, d2_ref, b_ref, o_ref):
    logits = p0_ref[...] + p1_ref[...] + d2_ref[...] * z_ref[...] + b_ref[...]
    col = lax.broadcasted_iota(_i32, (128, _CP), 1)
    valid = col < _C
    neg = jnp.float32(-1e30)
    m = jnp.max(jnp.where(valid, logits, neg), axis=-1, keepdims=True)
    ex = jnp.where(valid, jnp.exp(logits - m), 0.0)
    lse = jnp.log(jnp.sum(ex, axis=-1, keepdims=True))
    o_ref[...] = logits - m - lse


def _final_call(p0, p1, z, d2, b):
    return pl.pallas_call(
        _final_body,
        grid=(_N_PAD // 128,),
        in_specs=[pl.BlockSpec((128, _CP), lambda i: (i, 0)),
                  pl.BlockSpec((128, _CP), lambda i: (i, 0)),
                  pl.BlockSpec((128, _CP), lambda i: (i, 0)),
                  pl.BlockSpec((128, 1), lambda i: (i, 0)),
                  pl.BlockSpec((1, _CP), lambda i: (0, 0))],
        out_specs=pl.BlockSpec((128, _CP), lambda i: (i, 0)),
        out_shape=jax.ShapeDtypeStruct((_N_PAD, _CP), _f32),
    )(p0, p1, z, d2, b)


def kernel(x, edge_index, edge_attr, W1, b1, W2, b2, W3, b3):
    row = edge_index[0]
    col = edge_index[1]
    rpad = jnp.pad(row, (0, _E_PAD - _E))
    cpad = jnp.pad(col, (0, _E_PAD - _E))
    epad = jnp.pad(edge_attr, (0, _E_PAD - _E))
    row3 = rpad.reshape(_NW, _CH, _EB)
    col3 = cpad.reshape(_NW, _CH, _EB)
    ew3 = epad.reshape(_NW, _CH, _EB)
    rowf = rpad.reshape(_NW, _EPT)
    colf = cpad.reshape(_NW, _EPT)
    ewf = epad.reshape(_NW, _EPT)
    xp = jnp.pad(x, ((0, _N_PAD - _N), (0, 0)))
    W3p = jnp.pad(W3, ((0, 0), (0, _CP - _C)))
    b3p = jnp.pad(b3, (0, _CP - _C))

    deg_parts = _deg_call(col3, ew3)
    dinv, dinv2 = _dinv_call(deg_parts.reshape(_NC, _N_PAD, 1))
    z1 = _mm_call(xp, W1)
    norm = _norm_call(rowf, colf, ewf, dinv.reshape(_N_PAD))
    norm3 = norm.reshape(_NW, _CH, _EB)

    p1 = _agg128(z1, row3, col3, norm3)
    z2 = _epi_call(p1[0], p1[1], z1, dinv2, b1.reshape(1, _H), W2)
    p2 = _agg128(z2, row3, col3, norm3)
    z3 = _epi_call(p2[0], p2[1], z2, dinv2, b2.reshape(1, _H), W3p)
    p3 = _agg48(z3, row3, col3, norm3)
    out = _final_call(p3[0], p3[1], z3, dinv2, b3p.reshape(1, _CP))
    return out[:_N, :_C]


# speedup vs baseline: 1.0998x; 1.0998x over previous
import functools

import jax
import jax.numpy as jnp
from jax import lax
from jax.experimental import pallas as pl
from jax.experimental.pallas import tpu as pltpu
from jax.experimental.pallas import tpu_sc as plsc

_N = 10000
_E = 320000
_D = 128
_H = 128
_C = 40

_NC = 2
_NS = 16
_NW = _NC * _NS
_EB = 32
_CH = 320
_SEG = 160
_NB = 5
_EPT = _CH * _EB
_E_PAD = _NW * _EPT
_N_PAD = 10240
_RPT = _N_PAD // _NS
_CP = 48

_f32 = jnp.float32
_i32 = jnp.int32

_mesh = plsc.VectorSubcoreMesh(
    core_axis_name="c", subcore_axis_name="s",
    num_cores=_NC, num_subcores=_NS)

_sc_params = pltpu.CompilerParams(needs_layout_passes=False,
                                  use_tc_tiling_on_sc=False)


def _deg_body(col_hbm, ew_hbm, parts_hbm, cidx, ewv, zb, acc):
    c = lax.axis_index("c")
    s = lax.axis_index("s")
    w = c * _NS + s
    for t in range(_RPT // 16):
        zb[pl.ds(t * 16, 16)] = jnp.zeros((16,), _f32)
    pltpu.sync_copy(zb, acc.at[pl.ds(s * _RPT, _RPT)])
    plsc.subcore_barrier()
    pltpu.sync_copy(col_hbm.at[w], cidx)
    pltpu.sync_copy(ew_hbm.at[w], ewv)

    def j_body(j, carry):
        pltpu.sync_copy(ewv.at[j], acc.at[cidx.at[j]], add=True)
        return carry

    lax.fori_loop(0, _CH, j_body, 0)
    plsc.subcore_barrier()
    pltpu.sync_copy(acc.at[pl.ds(s * _RPT, _RPT)],
                    parts_hbm.at[c, pl.ds(s * _RPT, _RPT)])


_deg_call = pl.kernel(
    _deg_body,
    out_type=jax.ShapeDtypeStruct((_NC, _N_PAD), _f32),
    mesh=_mesh,
    compiler_params=_sc_params,
    scratch_types=[
        pltpu.VMEM((_CH, _EB), _i32),
        pltpu.VMEM((_CH, _EB), _f32),
        pltpu.VMEM((_RPT,), _f32),
        pltpu.VMEM_SHARED((_N_PAD,), _f32),
    ],
)


def _norm_body(row_hbm, col_hbm, ew_hbm, dinv_hbm, norm_hbm, dv, rv, cv, ev, nv):
    c = lax.axis_index("c")
    s = lax.axis_index("s")
    w = c * _NS + s
    pltpu.sync_copy(dinv_hbm, dv)
    pltpu.sync_copy(row_hbm.at[w], rv)
    pltpu.sync_copy(col_hbm.at[w], cv)
    pltpu.sync_copy(ew_hbm.at[w], ev)

    def t_body(t, carry):
        r16 = rv[pl.ds(t * 16, 16)]
        c16 = cv[pl.ds(t * 16, 16)]
        e16 = ev[pl.ds(t * 16, 16)]
        n16 = plsc.load_gather(dv, [r16]) * e16 * plsc.load_gather(dv, [c16])
        nv[pl.ds(t * 16, 16)] = n16
        return carry

    lax.fori_loop(0, _EPT // 16, t_body, 0)
    pltpu.sync_copy(nv, norm_hbm.at[w])


_norm_call = pl.kernel(
    _norm_body,
    out_type=jax.ShapeDtypeStruct((_NW, _EPT), _f32),
    mesh=_mesh,
    compiler_params=_sc_params,
    scratch_types=[
        pltpu.VMEM((_N_PAD,), _f32),
        pltpu.VMEM((_EPT,), _i32),
        pltpu.VMEM((_EPT,), _i32),
        pltpu.VMEM((_EPT,), _f32),
        pltpu.VMEM((_EPT,), _f32),
    ],
)


def _agg_body(Wd, spmem_z, z_hbm, row_hbm, col_hbm, norm_hbm, parts_hbm,
              *refs):
    if spmem_z:
        (ridx, cidx, nv, gb0, gb1, gb2, gb3, gb4, acc, zsh,
         gs0, gs1, gs2, gs3, gs4, ss0, ss1, ss2, ss3, ss4) = refs
    else:
        (ridx, cidx, nv, gb0, gb1, gb2, gb3, gb4, acc,
         gs0, gs1, gs2, gs3, gs4, ss0, ss1, ss2, ss3, ss4) = refs
        zsh = None
    c = lax.axis_index("c")
    s = lax.axis_index("s")
    w = c * _NS + s
    gbs = (gb0, gb1, gb2, gb3, gb4)
    gss = (gs0, gs1, gs2, gs3, gs4)
    sss = (ss0, ss1, ss2, ss3, ss4)

    if spmem_z:
        pltpu.sync_copy(z_hbm.at[pl.ds(s * _RPT, _RPT)],
                        zsh.at[pl.ds(s * _RPT, _RPT)])
    zsrc = zsh if spmem_z else z_hbm

    def zrow(r, carry):
        for v in range(Wd // 16):
            gb0[r, pl.ds(v * 16, 16)] = jnp.zeros((16,), _f32)
        return carry

    lax.fori_loop(0, _EB, zrow, 0)

    def zcopy(k, carry):
        pltpu.sync_copy(gb0, acc.at[pl.ds(s * _RPT + k * _EB, _EB)])
        return carry

    lax.fori_loop(0, _RPT // _EB, zcopy, 0)
    plsc.subcore_barrier()

    def stage(seg):
        base = seg * _SEG
        pltpu.sync_copy(row_hbm.at[w, pl.ds(base, _SEG)], ridx)
        pltpu.sync_copy(col_hbm.at[w, pl.ds(base, _SEG)], cidx)
        pltpu.sync_copy(norm_hbm.at[w, pl.ds(base, _SEG)], nv)

    def start_gather(l, b):
        pltpu.async_copy(zsrc.at[ridx.at[l]], gbs[b], gss[b])

    def wait_gather(l, b):
        pltpu.make_async_copy(zsrc.at[ridx.at[l]], gbs[b], gss[b]).wait()

    def scale(l, b):
        gb = gbs[b]

        @plsc.parallel_loop(0, _EB, 1, unroll=4)
        def _(e):
            nspl = plsc.load_gather(
                nv, [jnp.full((16,), l, _i32), jnp.full((16,), e, _i32)])
            for v in range(Wd // 16):
                sl = pl.ds(v * 16, 16)
                gb[e, sl] = gb[e, sl] * nspl

    def start_scatter(l, b):
        pltpu.async_copy(gbs[b], acc.at[cidx.at[l]], sss[b], add=True)

    def wait_scatter(l, b):
        pltpu.make_async_copy(gbs[b], acc.at[cidx.at[l]], sss[b]).wait()

    def run_seg():
        for b in range(3):
            start_gather(b, b)

        def k_body(k, carry):
            for phase in range(_NB):
                l = k * _NB + phase
                wait_gather(l, phase)
                scale(l, phase)
                start_scatter(l, phase)
                pb = (phase + 3) % _NB

                @pl.when(l >= 2)
                def _():
                    wait_scatter(l - 2, pb)

                @pl.when(l + 3 < _SEG)
                def _():
                    start_gather(l + 3, pb)

            return carry

        lax.fori_loop(0, _SEG // _NB, k_body, 0)
        wait_scatter(_SEG - 2, (_SEG - 2) % _NB)
        wait_scatter(_SEG - 1, (_SEG - 1) % _NB)

    stage(0)
    run_seg()
    stage(1)
    run_seg()
    plsc.subcore_barrier()

    def wb(k, carry):
        pltpu.sync_copy(acc.at[pl.ds(s * _RPT + k * 128, 128)],
                        parts_hbm.at[c, pl.ds(s * _RPT + k * 128, 128)])
        return carry

    lax.fori_loop(0, _RPT // 128, wb, 0)


def _make_agg(Wd, spmem_z):
    return pl.kernel(
        functools.partial(_agg_body, Wd, spmem_z),
        out_type=jax.ShapeDtypeStruct((_NC, _N_PAD, Wd), _f32),
        mesh=_mesh,
        compiler_params=_sc_params,
        scratch_types=(
            [pltpu.VMEM((_SEG, _EB), _i32),
             pltpu.VMEM((_SEG, _EB), _i32),
             pltpu.VMEM((_SEG, _EB), _f32)]
            + [pltpu.VMEM((_EB, Wd), _f32)] * _NB
            + [pltpu.VMEM_SHARED((_N_PAD, Wd), _f32)]
            + ([pltpu.VMEM_SHARED((_N_PAD, Wd), _f32)] if spmem_z else [])
            + [pltpu.SemaphoreType.DMA] * (2 * _NB)
        ),
    )


_agg128 = _make_agg(_H, False)
_agg48 = _make_agg(_CP, True)


def _dinv_body(parts_ref, dinv_ref, dinv2_ref):
    deg = 1.0 + parts_ref[0] + parts_ref[1]
    dv = jnp.where(deg > 0, lax.rsqrt(deg), 0.0)
    dinv_ref[...] = dv
    dinv2_ref[...] = dv * dv


def _dinv_call(parts):
    return pl.pallas_call(
        _dinv_body,
        grid=(_N_PAD // 128,),
        in_specs=[pl.BlockSpec((_NC, 128, 1), lambda i: (0, i, 0))],
        out_specs=[pl.BlockSpec((128, 1), lambda i: (i, 0)),
                   pl.BlockSpec((128, 1), lambda i: (i, 0))],
        out_shape=[jax.ShapeDtypeStruct((_N_PAD, 1), _f32),
                   jax.ShapeDtypeStruct((_N_PAD, 1), _f32)],
    )(parts)


def _mm_body(x_ref, w_ref, o_ref):
    o_ref[...] = jnp.dot(x_ref[...], w_ref[...],
                         preferred_element_type=_f32)


def _mm_call(x, w):
    wo = w.shape[1]
    return pl.pallas_call(
        _mm_body,
        grid=(_N_PAD // 128,),
        in_specs=[pl.BlockSpec((128, _D), lambda i: (i, 0)),
                  pl.BlockSpec((_D, wo), lambda i: (0, 0))],
        out_specs=pl.BlockSpec((128, wo), lambda i: (i, 0)),
        out_shape=jax.ShapeDtypeStruct((_N_PAD, wo), _f32),
    )(x, w)


def _epi_body(p0_ref, p1_ref, z_ref, d2_ref, b_ref, w_ref, o_ref):
    h = p0_ref[...] + p1_ref[...] + d2_ref[...] * z_ref[...] + b_ref[...]
    h = jnp.maximum(h, 0.0)
    o_ref[...] = jnp.dot(h, w_ref[...], preferred_element_type=_f32)


def _epi_call(p0, p1, z, d2, b, w):
    wo = w.shape[1]
    return pl.pallas_call(
        _epi_body,
        grid=(_N_PAD // 128,),
        in_specs=[pl.BlockSpec((128, _H), lambda i: (i, 0)),
                  pl.BlockSpec((128, _H), lambda i: (i, 0)),
                  pl.BlockSpec((128, _H), lambda i: (i, 0)),
                  pl.BlockSpec((128, 1), lambda i: (i, 0)),
                  pl.BlockSpec((1, _H), lambda i: (0, 0)),
                  pl.BlockSpec((_H, wo), lambda i: (0, 0))],
        out_specs=pl.BlockSpec((128, wo), lambda i: (i, 0)),
        out_shape=jax.ShapeDtypeStruct((_N_PAD, wo), _f32),
    )(p0, p1, z, d2, b, w)


def _final_body(p0_ref, p1_ref, z_ref, d2_ref, b_ref, o_ref):
    logits = p0_ref[...] + p1_ref[...] + d2_ref[...] * z_ref[...] + b_ref[...]
    col = lax.broadcasted_iota(_i32, (128, _CP), 1)
    valid = col < _C
    neg = jnp.float32(-1e30)
    m = jnp.max(jnp.where(valid, logits, neg), axis=-1, keepdims=True)
    ex = jnp.where(valid, jnp.exp(logits - m), 0.0)
    lse = jnp.log(jnp.sum(ex, axis=-1, keepdims=True))
    o_ref[...] = logits - m - lse


def _final_call(p0, p1, z, d2, b):
    return pl.pallas_call(
        _final_body,
        grid=(_N_PAD // 128,),
        in_specs=[pl.BlockSpec((128, _CP), lambda i: (i, 0)),
                  pl.BlockSpec((128, _CP), lambda i: (i, 0)),
                  pl.BlockSpec((128, _CP), lambda i: (i, 0)),
                  pl.BlockSpec((128, 1), lambda i: (i, 0)),
                  pl.BlockSpec((1, _CP), lambda i: (0, 0))],
        out_specs=pl.BlockSpec((128, _CP), lambda i: (i, 0)),
        out_shape=jax.ShapeDtypeStruct((_N_PAD, _CP), _f32),
    )(p0, p1, z, d2, b)


def kernel(x, edge_index, edge_attr, W1, b1, W2, b2, W3, b3):
    row = edge_index[0]
    col = edge_index[1]
    rpad = jnp.pad(row, (0, _E_PAD - _E))
    cpad = jnp.pad(col, (0, _E_PAD - _E))
    epad = jnp.pad(edge_attr, (0, _E_PAD - _E))
    row3 = rpad.reshape(_NW, _CH, _EB)
    col3 = cpad.reshape(_NW, _CH, _EB)
    ew3 = epad.reshape(_NW, _CH, _EB)
    rowf = rpad.reshape(_NW, _EPT)
    colf = cpad.reshape(_NW, _EPT)
    ewf = epad.reshape(_NW, _EPT)
    xp = jnp.pad(x, ((0, _N_PAD - _N), (0, 0)))
    W3p = jnp.pad(W3, ((0, 0), (0, _CP - _C)))
    b3p = jnp.pad(b3, (0, _CP - _C))

    deg_parts = _deg_call(col3, ew3)
    dinv, dinv2 = _dinv_call(deg_parts.reshape(_NC, _N_PAD, 1))
    z1 = _mm_call(xp, W1)
    norm = _norm_call(rowf, colf, ewf, dinv.reshape(_N_PAD))
    norm3 = norm.reshape(_NW, _CH, _EB)

    p1 = _agg128(z1, row3, col3, norm3)
    z2 = _epi_call(p1[0], p1[1], z1, dinv2, b1.reshape(1, _H), W2)
    p2 = _agg128(z2, row3, col3, norm3)
    z3 = _epi_call(p2[0], p2[1], z2, dinv2, b2.reshape(1, _H), W3p)
    p3 = _agg48(z3, row3, col3, norm3)
    out = _final_call(p3[0], p3[1], z3, dinv2, b3p.reshape(1, _CP))
    return out[:_N, :_C]


# speedup vs baseline: 1.8794x; 1.7088x over previous
import functools

import jax
import jax.numpy as jnp
from jax import lax
from jax.experimental import pallas as pl
from jax.experimental.pallas import tpu as pltpu
from jax.experimental.pallas import tpu_sc as plsc

_N = 10000
_E = 320000
_D = 128
_H = 128
_C = 40

_NC = 2
_NS = 16
_NW = _NC * _NS
_EB = 32
_CH = 320
_SEG = 160
_NB = 5
_EPT = _CH * _EB
_E_PAD = _NW * _EPT
_N_PAD = 10240
_RPT = _N_PAD // _NS
_CP = 48

_f32 = jnp.float32
_i32 = jnp.int32

_mesh = plsc.VectorSubcoreMesh(
    core_axis_name="c", subcore_axis_name="s",
    num_cores=_NC, num_subcores=_NS)

_sc_params = pltpu.CompilerParams(needs_layout_passes=False,
                                  use_tc_tiling_on_sc=False)


def _deg_body(col_hbm, ew_hbm, parts_hbm, cidx, ewv, zb, acc):
    c = lax.axis_index("c")
    s = lax.axis_index("s")
    w = c * _NS + s
    for t in range(_RPT // 16):
        zb[pl.ds(t * 16, 16)] = jnp.zeros((16,), _f32)
    pltpu.sync_copy(zb, acc.at[pl.ds(s * _RPT, _RPT)])
    plsc.subcore_barrier()
    pltpu.sync_copy(col_hbm.at[w], cidx)
    pltpu.sync_copy(ew_hbm.at[w], ewv)

    def j_body(j, carry):
        pltpu.sync_copy(ewv.at[j], acc.at[cidx.at[j]], add=True)
        return carry

    lax.fori_loop(0, _CH, j_body, 0)
    plsc.subcore_barrier()
    pltpu.sync_copy(acc.at[pl.ds(s * _RPT, _RPT)],
                    parts_hbm.at[c, pl.ds(s * _RPT, _RPT)])


_deg_call = pl.kernel(
    _deg_body,
    out_type=jax.ShapeDtypeStruct((_NC, _N_PAD), _f32),
    mesh=_mesh,
    compiler_params=_sc_params,
    scratch_types=[
        pltpu.VMEM((_CH, _EB), _i32),
        pltpu.VMEM((_CH, _EB), _f32),
        pltpu.VMEM((_RPT,), _f32),
        pltpu.VMEM_SHARED((_N_PAD,), _f32),
    ],
)


def _norm_body(row_hbm, col_hbm, ew_hbm, dinv_hbm, norm_hbm, dv, rv, cv, ev, nv):
    c = lax.axis_index("c")
    s = lax.axis_index("s")
    w = c * _NS + s
    pltpu.sync_copy(dinv_hbm, dv)
    pltpu.sync_copy(row_hbm.at[w], rv)
    pltpu.sync_copy(col_hbm.at[w], cv)
    pltpu.sync_copy(ew_hbm.at[w], ev)

    def t_body(t, carry):
        r16 = rv[pl.ds(t * 16, 16)]
        c16 = cv[pl.ds(t * 16, 16)]
        e16 = ev[pl.ds(t * 16, 16)]
        n16 = plsc.load_gather(dv, [r16]) * e16 * plsc.load_gather(dv, [c16])
        nv[pl.ds(t * 16, 16)] = n16
        return carry

    lax.fori_loop(0, _EPT // 16, t_body, 0)
    pltpu.sync_copy(nv, norm_hbm.at[w])


_norm_call = pl.kernel(
    _norm_body,
    out_type=jax.ShapeDtypeStruct((_NW, _EPT), _f32),
    mesh=_mesh,
    compiler_params=_sc_params,
    scratch_types=[
        pltpu.VMEM((_N_PAD,), _f32),
        pltpu.VMEM((_EPT,), _i32),
        pltpu.VMEM((_EPT,), _i32),
        pltpu.VMEM((_EPT,), _f32),
        pltpu.VMEM((_EPT,), _f32),
    ],
)


def _agg_body(Wd, spmem_z, z_hbm, row_hbm, col_hbm, norm_hbm, parts_hbm,
              *refs):
    if spmem_z:
        (ridx, cidx, nv, gb0, gb1, gb2, gb3, gb4, acc, zsh,
         gs0, gs1, gs2, gs3, gs4, ss0, ss1, ss2, ss3, ss4) = refs
    else:
        (ridx, cidx, nv, gb0, gb1, gb2, gb3, gb4, acc,
         gs0, gs1, gs2, gs3, gs4, ss0, ss1, ss2, ss3, ss4) = refs
        zsh = None
    c = lax.axis_index("c")
    s = lax.axis_index("s")
    w = c * _NS + s
    gbs = (gb0, gb1, gb2, gb3, gb4)
    gss = (gs0, gs1, gs2, gs3, gs4)
    sss = (ss0, ss1, ss2, ss3, ss4)

    if spmem_z:
        pltpu.sync_copy(z_hbm.at[pl.ds(s * _RPT, _RPT)],
                        zsh.at[pl.ds(s * _RPT, _RPT)])
    zsrc = zsh if spmem_z else z_hbm

    def zrow(r, carry):
        for v in range(Wd // 16):
            gb0[r, pl.ds(v * 16, 16)] = jnp.zeros((16,), _f32)
        return carry

    lax.fori_loop(0, _EB, zrow, 0)

    def zcopy(k, carry):
        pltpu.sync_copy(gb0, acc.at[pl.ds(s * _RPT + k * _EB, _EB)])
        return carry

    lax.fori_loop(0, _RPT // _EB, zcopy, 0)
    plsc.subcore_barrier()

    def stage(seg):
        base = seg * _SEG
        pltpu.sync_copy(row_hbm.at[w, pl.ds(base, _SEG)], ridx)
        pltpu.sync_copy(col_hbm.at[w, pl.ds(base, _SEG)], cidx)
        pltpu.sync_copy(norm_hbm.at[w, pl.ds(base, _SEG)], nv)

    def start_gather(l, b):
        pltpu.async_copy(zsrc.at[ridx.at[l]], gbs[b], gss[b])

    def wait_gather(l, b):
        pltpu.make_async_copy(zsrc.at[ridx.at[l]], gbs[b], gss[b]).wait()

    def scale(l, b):
        gb = gbs[b]

        @plsc.parallel_loop(0, _EB, 1, unroll=4)
        def _(e):
            nspl = plsc.load_gather(
                nv, [jnp.full((16,), l, _i32), jnp.full((16,), e, _i32)])
            for v in range(Wd // 16):
                sl = pl.ds(v * 16, 16)
                gb[e, sl] = gb[e, sl] * nspl

    def start_scatter(l, b):
        pltpu.async_copy(gbs[b], acc.at[cidx.at[l]], sss[b], add=True)

    def wait_scatter(l, b):
        pltpu.make_async_copy(gbs[b], acc.at[cidx.at[l]], sss[b]).wait()

    def run_seg():
        for b in range(3):
            start_gather(b, b)

        def k_body(k, carry):
            for phase in range(_NB):
                l = k * _NB + phase
                wait_gather(l, phase)
                scale(l, phase)
                start_scatter(l, phase)
                pb = (phase + 3) % _NB

                @pl.when(l >= 2)
                def _():
                    wait_scatter(l - 2, pb)

                @pl.when(l + 3 < _SEG)
                def _():
                    start_gather(l + 3, pb)

            return carry

        lax.fori_loop(0, _SEG // _NB, k_body, 0)
        wait_scatter(_SEG - 2, (_SEG - 2) % _NB)
        wait_scatter(_SEG - 1, (_SEG - 1) % _NB)

    stage(0)
    run_seg()
    stage(1)
    run_seg()
    plsc.subcore_barrier()

    def wb(k, carry):
        pltpu.sync_copy(acc.at[pl.ds(s * _RPT + k * 128, 128)],
                        parts_hbm.at[c, pl.ds(s * _RPT + k * 128, 128)])
        return carry

    lax.fori_loop(0, _RPT // 128, wb, 0)


def _make_agg(Wd, spmem_z):
    return pl.kernel(
        functools.partial(_agg_body, Wd, spmem_z),
        out_type=jax.ShapeDtypeStruct((_NC, _N_PAD, Wd), _f32),
        mesh=_mesh,
        compiler_params=_sc_params,
        scratch_types=(
            [pltpu.VMEM((_SEG, _EB), _i32),
             pltpu.VMEM((_SEG, _EB), _i32),
             pltpu.VMEM((_SEG, _EB), _f32)]
            + [pltpu.VMEM((_EB, Wd), _f32)] * _NB
            + [pltpu.VMEM_SHARED((_N_PAD, Wd), _f32)]
            + ([pltpu.VMEM_SHARED((_N_PAD, Wd), _f32)] if spmem_z else [])
            + [pltpu.SemaphoreType.DMA] * (2 * _NB)
        ),
    )


_agg48 = _make_agg(_CP, True)
_WH = 64


def _agg2_body(zlo_hbm, zhi_hbm, row_hbm, col_hbm, norm_hbm, parts_hbm,
               ridx, cidx, nv,
               gb0, gb1, gb2, gb3, gb4, acc, zsh,
               gs0, gs1, gs2, gs3, gs4,
               ss0, ss1, ss2, ss3, ss4):
    c = lax.axis_index("c")
    s = lax.axis_index("s")
    w = c * _NS + s
    gbs = (gb0, gb1, gb2, gb3, gb4)
    gss = (gs0, gs1, gs2, gs3, gs4)
    sss = (ss0, ss1, ss2, ss3, ss4)
    zhalves = (zlo_hbm, zhi_hbm)

    def stage(seg):
        base = seg * _SEG
        pltpu.sync_copy(row_hbm.at[w, pl.ds(base, _SEG)], ridx)
        pltpu.sync_copy(col_hbm.at[w, pl.ds(base, _SEG)], cidx)
        pltpu.sync_copy(norm_hbm.at[w, pl.ds(base, _SEG)], nv)

    def start_gather(l, b):
        pltpu.async_copy(zsh.at[ridx.at[l]], gbs[b], gss[b])

    def wait_gather(l, b):
        pltpu.make_async_copy(zsh.at[ridx.at[l]], gbs[b], gss[b]).wait()

    def scale(l, b):
        gb = gbs[b]

        @plsc.parallel_loop(0, _EB, 1, unroll=4)
        def _(e):
            nspl = plsc.load_gather(
                nv, [jnp.full((16,), l, _i32), jnp.full((16,), e, _i32)])
            for v in range(_WH // 16):
                sl = pl.ds(v * 16, 16)
                gb[e, sl] = gb[e, sl] * nspl

    def start_scatter(l, b):
        pltpu.async_copy(gbs[b], acc.at[cidx.at[l]], sss[b], add=True)

    def wait_scatter(l, b):
        pltpu.make_async_copy(gbs[b], acc.at[cidx.at[l]], sss[b]).wait()

    def run_seg():
        for b in range(3):
            start_gather(b, b)

        def k_body(k, carry):
            for phase in range(_NB):
                l = k * _NB + phase
                wait_gather(l, phase)
                scale(l, phase)
                start_scatter(l, phase)
                pb = (phase + 3) % _NB

                @pl.when(l >= 2)
                def _():
                    wait_scatter(l - 2, pb)

                @pl.when(l + 3 < _SEG)
                def _():
                    start_gather(l + 3, pb)

            return carry

        lax.fori_loop(0, _SEG // _NB, k_body, 0)
        wait_scatter(_SEG - 2, (_SEG - 2) % _NB)
        wait_scatter(_SEG - 1, (_SEG - 1) % _NB)

    for half in range(2):
        pltpu.sync_copy(zhalves[half].at[pl.ds(s * _RPT, _RPT)],
                        zsh.at[pl.ds(s * _RPT, _RPT)])

        def zrow(r, carry):
            for v in range(_WH // 16):
                gb0[r, pl.ds(v * 16, 16)] = jnp.zeros((16,), _f32)
            return carry

        lax.fori_loop(0, _EB, zrow, 0)

        def zcopy(k, carry):
            pltpu.sync_copy(gb0, acc.at[pl.ds(s * _RPT + k * _EB, _EB)])
            return carry

        lax.fori_loop(0, _RPT // _EB, zcopy, 0)
        plsc.subcore_barrier()

        stage(0)
        run_seg()
        stage(1)
        run_seg()
        plsc.subcore_barrier()

        def wb(k, carry):
            pltpu.sync_copy(
                acc.at[pl.ds(s * _RPT + k * 128, 128)],
                parts_hbm.at[half, c, pl.ds(s * _RPT + k * 128, 128)])
            return carry

        lax.fori_loop(0, _RPT // 128, wb, 0)


_agg2 = pl.kernel(
    _agg2_body,
    out_type=jax.ShapeDtypeStruct((2, _NC, _N_PAD, _WH), _f32),
    mesh=_mesh,
    compiler_params=_sc_params,
    scratch_types=(
        [pltpu.VMEM((_SEG, _EB), _i32),
         pltpu.VMEM((_SEG, _EB), _i32),
         pltpu.VMEM((_SEG, _EB), _f32)]
        + [pltpu.VMEM((_EB, _WH), _f32)] * _NB
        + [pltpu.VMEM_SHARED((_N_PAD, _WH), _f32)] * 2
        + [pltpu.SemaphoreType.DMA] * (2 * _NB)
    ),
)


def _dinv_body(parts_ref, dinv_ref, dinv2_ref):
    deg = 1.0 + parts_ref[0] + parts_ref[1]
    dv = jnp.where(deg > 0, lax.rsqrt(deg), 0.0)
    dinv_ref[...] = dv
    dinv2_ref[...] = dv * dv


def _dinv_call(parts):
    return pl.pallas_call(
        _dinv_body,
        grid=(_N_PAD // 128,),
        in_specs=[pl.BlockSpec((_NC, 128, 1), lambda i: (0, i, 0))],
        out_specs=[pl.BlockSpec((128, 1), lambda i: (i, 0)),
                   pl.BlockSpec((128, 1), lambda i: (i, 0))],
        out_shape=[jax.ShapeDtypeStruct((_N_PAD, 1), _f32),
                   jax.ShapeDtypeStruct((_N_PAD, 1), _f32)],
    )(parts)


def _mm_body(x_ref, w_ref, o_ref):
    o_ref[...] = jnp.dot(x_ref[...], w_ref[...],
                         preferred_element_type=_f32)


def _mm_call(x, w):
    wo = w.shape[1]
    return pl.pallas_call(
        _mm_body,
        grid=(_N_PAD // 128,),
        in_specs=[pl.BlockSpec((128, _D), lambda i: (i, 0)),
                  pl.BlockSpec((_D, wo), lambda i: (0, 0))],
        out_specs=pl.BlockSpec((128, wo), lambda i: (i, 0)),
        out_shape=jax.ShapeDtypeStruct((_N_PAD, wo), _f32),
    )(x, w)


def _epi_body(p_ref, z_ref, d2_ref, b_ref, w_ref, o_ref):
    agg = jnp.concatenate(
        [p_ref[0, 0] + p_ref[0, 1], p_ref[1, 0] + p_ref[1, 1]], axis=-1)
    h = agg + d2_ref[...] * z_ref[...] + b_ref[...]
    h = jnp.maximum(h, 0.0)
    o_ref[...] = jnp.dot(h, w_ref[...], preferred_element_type=_f32)


def _epi_call(p, z, d2, b, w):
    wo = w.shape[1]
    return pl.pallas_call(
        _epi_body,
        grid=(_N_PAD // 128,),
        in_specs=[pl.BlockSpec((2, _NC, 128, _WH), lambda i: (0, 0, i, 0)),
                  pl.BlockSpec((128, _H), lambda i: (i, 0)),
                  pl.BlockSpec((128, 1), lambda i: (i, 0)),
                  pl.BlockSpec((1, _H), lambda i: (0, 0)),
                  pl.BlockSpec((_H, wo), lambda i: (0, 0))],
        out_specs=pl.BlockSpec((128, wo), lambda i: (i, 0)),
        out_shape=jax.ShapeDtypeStruct((_N_PAD, wo), _f32),
    )(p, z, d2, b, w)


def _final_body(p0_ref, p1_ref, z_ref, d2_ref, b_ref, o_ref):
    logits = p0_ref[...] + p1_ref[...] + d2_ref[...] * z_ref[...] + b_ref[...]
    col = lax.broadcasted_iota(_i32, (128, _CP), 1)
    valid = col < _C
    neg = jnp.float32(-1e30)
    m = jnp.max(jnp.where(valid, logits, neg), axis=-1, keepdims=True)
    ex = jnp.where(valid, jnp.exp(logits - m), 0.0)
    lse = jnp.log(jnp.sum(ex, axis=-1, keepdims=True))
    o_ref[...] = logits - m - lse


def _final_call(p0, p1, z, d2, b):
    return pl.pallas_call(
        _final_body,
        grid=(_N_PAD // 128,),
        in_specs=[pl.BlockSpec((128, _CP), lambda i: (i, 0)),
                  pl.BlockSpec((128, _CP), lambda i: (i, 0)),
                  pl.BlockSpec((128, _CP), lambda i: (i, 0)),
                  pl.BlockSpec((128, 1), lambda i: (i, 0)),
                  pl.BlockSpec((1, _CP), lambda i: (0, 0))],
        out_specs=pl.BlockSpec((128, _CP), lambda i: (i, 0)),
        out_shape=jax.ShapeDtypeStruct((_N_PAD, _CP), _f32),
    )(p0, p1, z, d2, b)


def kernel(x, edge_index, edge_attr, W1, b1, W2, b2, W3, b3):
    row = edge_index[0]
    col = edge_index[1]
    rpad = jnp.pad(row, (0, _E_PAD - _E))
    cpad = jnp.pad(col, (0, _E_PAD - _E))
    epad = jnp.pad(edge_attr, (0, _E_PAD - _E))
    row3 = rpad.reshape(_NW, _CH, _EB)
    col3 = cpad.reshape(_NW, _CH, _EB)
    ew3 = epad.reshape(_NW, _CH, _EB)
    rowf = rpad.reshape(_NW, _EPT)
    colf = cpad.reshape(_NW, _EPT)
    ewf = epad.reshape(_NW, _EPT)
    xp = jnp.pad(x, ((0, _N_PAD - _N), (0, 0)))
    W3p = jnp.pad(W3, ((0, 0), (0, _CP - _C)))
    b3p = jnp.pad(b3, (0, _CP - _C))

    deg_parts = _deg_call(col3, ew3)
    dinv, dinv2 = _dinv_call(deg_parts.reshape(_NC, _N_PAD, 1))
    z1 = _mm_call(xp, W1)
    norm = _norm_call(rowf, colf, ewf, dinv.reshape(_N_PAD))
    norm3 = norm.reshape(_NW, _CH, _EB)

    p1 = _agg2(z1[:, :_WH], z1[:, _WH:], row3, col3, norm3)
    z2 = _epi_call(p1, z1, dinv2, b1.reshape(1, _H), W2)
    p2 = _agg2(z2[:, :_WH], z2[:, _WH:], row3, col3, norm3)
    z3 = _epi_call(p2, z2, dinv2, b2.reshape(1, _H), W3p)
    p3 = _agg48(z3, row3, col3, norm3)
    out = _final_call(p3[0], p3[1], z3, dinv2, b3p.reshape(1, _CP))
    return out[:_N, :_C]


# speedup vs baseline: 1.8927x; 1.0070x over previous
import functools

import jax
import jax.numpy as jnp
from jax import lax
from jax.experimental import pallas as pl
from jax.experimental.pallas import tpu as pltpu
from jax.experimental.pallas import tpu_sc as plsc

_N = 10000
_E = 320000
_D = 128
_H = 128
_C = 40

_NC = 2
_NS = 16
_NW = _NC * _NS
_EB = 64
_CH = 160
_SEG = 80
_NB = 5
_EPT = _CH * _EB
_E_PAD = _NW * _EPT
_N_PAD = 10240
_RPT = _N_PAD // _NS
_CP = 48

_f32 = jnp.float32
_i32 = jnp.int32

_mesh = plsc.VectorSubcoreMesh(
    core_axis_name="c", subcore_axis_name="s",
    num_cores=_NC, num_subcores=_NS)

_sc_params = pltpu.CompilerParams(needs_layout_passes=False,
                                  use_tc_tiling_on_sc=False)


def _deg_body(col_hbm, ew_hbm, parts_hbm, cidx, ewv, zb, acc):
    c = lax.axis_index("c")
    s = lax.axis_index("s")
    w = c * _NS + s
    for t in range(_RPT // 16):
        zb[pl.ds(t * 16, 16)] = jnp.zeros((16,), _f32)
    pltpu.sync_copy(zb, acc.at[pl.ds(s * _RPT, _RPT)])
    plsc.subcore_barrier()
    pltpu.sync_copy(col_hbm.at[w], cidx)
    pltpu.sync_copy(ew_hbm.at[w], ewv)

    def j_body(j, carry):
        pltpu.sync_copy(ewv.at[j], acc.at[cidx.at[j]], add=True)
        return carry

    lax.fori_loop(0, _CH, j_body, 0)
    plsc.subcore_barrier()
    pltpu.sync_copy(acc.at[pl.ds(s * _RPT, _RPT)],
                    parts_hbm.at[c, pl.ds(s * _RPT, _RPT)])


_deg_call = pl.kernel(
    _deg_body,
    out_type=jax.ShapeDtypeStruct((_NC, _N_PAD), _f32),
    mesh=_mesh,
    compiler_params=_sc_params,
    scratch_types=[
        pltpu.VMEM((_CH, _EB), _i32),
        pltpu.VMEM((_CH, _EB), _f32),
        pltpu.VMEM((_RPT,), _f32),
        pltpu.VMEM_SHARED((_N_PAD,), _f32),
    ],
)


def _norm_body(row_hbm, col_hbm, ew_hbm, dinv_hbm, norm_hbm, dv, rv, cv, ev, nv):
    c = lax.axis_index("c")
    s = lax.axis_index("s")
    w = c * _NS + s
    pltpu.sync_copy(dinv_hbm, dv)
    pltpu.sync_copy(row_hbm.at[w], rv)
    pltpu.sync_copy(col_hbm.at[w], cv)
    pltpu.sync_copy(ew_hbm.at[w], ev)

    def t_body(t, carry):
        r16 = rv[pl.ds(t * 16, 16)]
        c16 = cv[pl.ds(t * 16, 16)]
        e16 = ev[pl.ds(t * 16, 16)]
        n16 = plsc.load_gather(dv, [r16]) * e16 * plsc.load_gather(dv, [c16])
        nv[pl.ds(t * 16, 16)] = n16
        return carry

    lax.fori_loop(0, _EPT // 16, t_body, 0)
    pltpu.sync_copy(nv, norm_hbm.at[w])


_norm_call = pl.kernel(
    _norm_body,
    out_type=jax.ShapeDtypeStruct((_NW, _EPT), _f32),
    mesh=_mesh,
    compiler_params=_sc_params,
    scratch_types=[
        pltpu.VMEM((_N_PAD,), _f32),
        pltpu.VMEM((_EPT,), _i32),
        pltpu.VMEM((_EPT,), _i32),
        pltpu.VMEM((_EPT,), _f32),
        pltpu.VMEM((_EPT,), _f32),
    ],
)


def _agg_body(Wd, spmem_z, z_hbm, row_hbm, col_hbm, norm_hbm, parts_hbm,
              *refs):
    if spmem_z:
        (ridx, cidx, nv, gb0, gb1, gb2, gb3, gb4, acc, zsh,
         gs0, gs1, gs2, gs3, gs4, ss0, ss1, ss2, ss3, ss4) = refs
    else:
        (ridx, cidx, nv, gb0, gb1, gb2, gb3, gb4, acc,
         gs0, gs1, gs2, gs3, gs4, ss0, ss1, ss2, ss3, ss4) = refs
        zsh = None
    c = lax.axis_index("c")
    s = lax.axis_index("s")
    w = c * _NS + s
    gbs = (gb0, gb1, gb2, gb3, gb4)
    gss = (gs0, gs1, gs2, gs3, gs4)
    sss = (ss0, ss1, ss2, ss3, ss4)

    if spmem_z:
        pltpu.sync_copy(z_hbm.at[pl.ds(s * _RPT, _RPT)],
                        zsh.at[pl.ds(s * _RPT, _RPT)])
    zsrc = zsh if spmem_z else z_hbm

    def zrow(r, carry):
        for v in range(Wd // 16):
            gb0[r, pl.ds(v * 16, 16)] = jnp.zeros((16,), _f32)
        return carry

    lax.fori_loop(0, _EB, zrow, 0)

    def zcopy(k, carry):
        pltpu.sync_copy(gb0, acc.at[pl.ds(s * _RPT + k * _EB, _EB)])
        return carry

    lax.fori_loop(0, _RPT // _EB, zcopy, 0)
    plsc.subcore_barrier()

    def stage(seg):
        base = seg * _SEG
        pltpu.sync_copy(row_hbm.at[w, pl.ds(base, _SEG)], ridx)
        pltpu.sync_copy(col_hbm.at[w, pl.ds(base, _SEG)], cidx)
        pltpu.sync_copy(norm_hbm.at[w, pl.ds(base, _SEG)], nv)

    def start_gather(l, b):
        pltpu.async_copy(zsrc.at[ridx.at[l]], gbs[b], gss[b])

    def wait_gather(l, b):
        pltpu.make_async_copy(zsrc.at[ridx.at[l]], gbs[b], gss[b]).wait()

    def scale(l, b):
        gb = gbs[b]

        @plsc.parallel_loop(0, _EB, 1, unroll=4)
        def _(e):
            nspl = plsc.load_gather(
                nv, [jnp.full((16,), l, _i32), jnp.full((16,), e, _i32)])
            for v in range(Wd // 16):
                sl = pl.ds(v * 16, 16)
                gb[e, sl] = gb[e, sl] * nspl

    def start_scatter(l, b):
        pltpu.async_copy(gbs[b], acc.at[cidx.at[l]], sss[b], add=True)

    def wait_scatter(l, b):
        pltpu.make_async_copy(gbs[b], acc.at[cidx.at[l]], sss[b]).wait()

    def run_seg():
        for b in range(3):
            start_gather(b, b)

        def k_body(k, carry):
            for phase in range(_NB):
                l = k * _NB + phase
                wait_gather(l, phase)
                scale(l, phase)
                start_scatter(l, phase)
                pb = (phase + 3) % _NB

                @pl.when(l >= 2)
                def _():
                    wait_scatter(l - 2, pb)

                @pl.when(l + 3 < _SEG)
                def _():
                    start_gather(l + 3, pb)

            return carry

        lax.fori_loop(0, _SEG // _NB, k_body, 0)
        wait_scatter(_SEG - 2, (_SEG - 2) % _NB)
        wait_scatter(_SEG - 1, (_SEG - 1) % _NB)

    stage(0)
    run_seg()
    stage(1)
    run_seg()
    plsc.subcore_barrier()

    def wb(k, carry):
        pltpu.sync_copy(acc.at[pl.ds(s * _RPT + k * 128, 128)],
                        parts_hbm.at[c, pl.ds(s * _RPT + k * 128, 128)])
        return carry

    lax.fori_loop(0, _RPT // 128, wb, 0)


def _make_agg(Wd, spmem_z):
    return pl.kernel(
        functools.partial(_agg_body, Wd, spmem_z),
        out_type=jax.ShapeDtypeStruct((_NC, _N_PAD, Wd), _f32),
        mesh=_mesh,
        compiler_params=_sc_params,
        scratch_types=(
            [pltpu.VMEM((_SEG, _EB), _i32),
             pltpu.VMEM((_SEG, _EB), _i32),
             pltpu.VMEM((_SEG, _EB), _f32)]
            + [pltpu.VMEM((_EB, Wd), _f32)] * _NB
            + [pltpu.VMEM_SHARED((_N_PAD, Wd), _f32)]
            + ([pltpu.VMEM_SHARED((_N_PAD, Wd), _f32)] if spmem_z else [])
            + [pltpu.SemaphoreType.DMA] * (2 * _NB)
        ),
    )


_agg48 = _make_agg(_CP, True)
_WH = 64


def _agg2_body(zlo_hbm, zhi_hbm, row_hbm, col_hbm, norm_hbm, parts_hbm,
               ridx, cidx, nv,
               gb0, gb1, gb2, gb3, gb4, acc, zsh,
               gs0, gs1, gs2, gs3, gs4,
               ss0, ss1, ss2, ss3, ss4):
    c = lax.axis_index("c")
    s = lax.axis_index("s")
    w = c * _NS + s
    gbs = (gb0, gb1, gb2, gb3, gb4)
    gss = (gs0, gs1, gs2, gs3, gs4)
    sss = (ss0, ss1, ss2, ss3, ss4)
    zhalves = (zlo_hbm, zhi_hbm)

    def stage(seg):
        base = seg * _SEG
        pltpu.sync_copy(row_hbm.at[w, pl.ds(base, _SEG)], ridx)
        pltpu.sync_copy(col_hbm.at[w, pl.ds(base, _SEG)], cidx)
        pltpu.sync_copy(norm_hbm.at[w, pl.ds(base, _SEG)], nv)

    def start_gather(l, b):
        pltpu.async_copy(zsh.at[ridx.at[l]], gbs[b], gss[b])

    def wait_gather(l, b):
        pltpu.make_async_copy(zsh.at[ridx.at[l]], gbs[b], gss[b]).wait()

    def scale(l, b):
        gb = gbs[b]

        @plsc.parallel_loop(0, _EB, 1, unroll=4)
        def _(e):
            nspl = plsc.load_gather(
                nv, [jnp.full((16,), l, _i32), jnp.full((16,), e, _i32)])
            for v in range(_WH // 16):
                sl = pl.ds(v * 16, 16)
                gb[e, sl] = gb[e, sl] * nspl

    def start_scatter(l, b):
        pltpu.async_copy(gbs[b], acc.at[cidx.at[l]], sss[b], add=True)

    def wait_scatter(l, b):
        pltpu.make_async_copy(gbs[b], acc.at[cidx.at[l]], sss[b]).wait()

    def run_seg():
        for b in range(3):
            start_gather(b, b)

        def k_body(k, carry):
            for phase in range(_NB):
                l = k * _NB + phase
                wait_gather(l, phase)
                scale(l, phase)
                start_scatter(l, phase)
                pb = (phase + 3) % _NB

                @pl.when(l >= 2)
                def _():
                    wait_scatter(l - 2, pb)

                @pl.when(l + 3 < _SEG)
                def _():
                    start_gather(l + 3, pb)

            return carry

        lax.fori_loop(0, _SEG // _NB, k_body, 0)
        wait_scatter(_SEG - 2, (_SEG - 2) % _NB)
        wait_scatter(_SEG - 1, (_SEG - 1) % _NB)

    for half in range(2):
        pltpu.sync_copy(zhalves[half].at[pl.ds(s * _RPT, _RPT)],
                        zsh.at[pl.ds(s * _RPT, _RPT)])

        def zrow(r, carry):
            for v in range(_WH // 16):
                gb0[r, pl.ds(v * 16, 16)] = jnp.zeros((16,), _f32)
            return carry

        lax.fori_loop(0, _EB, zrow, 0)

        def zcopy(k, carry):
            pltpu.sync_copy(gb0, acc.at[pl.ds(s * _RPT + k * _EB, _EB)])
            return carry

        lax.fori_loop(0, _RPT // _EB, zcopy, 0)
        plsc.subcore_barrier()

        stage(0)
        run_seg()
        stage(1)
        run_seg()
        plsc.subcore_barrier()

        def wb(k, carry):
            pltpu.sync_copy(
                acc.at[pl.ds(s * _RPT + k * 128, 128)],
                parts_hbm.at[half, c, pl.ds(s * _RPT + k * 128, 128)])
            return carry

        lax.fori_loop(0, _RPT // 128, wb, 0)


_agg2 = pl.kernel(
    _agg2_body,
    out_type=jax.ShapeDtypeStruct((2, _NC, _N_PAD, _WH), _f32),
    mesh=_mesh,
    compiler_params=_sc_params,
    scratch_types=(
        [pltpu.VMEM((_SEG, _EB), _i32),
         pltpu.VMEM((_SEG, _EB), _i32),
         pltpu.VMEM((_SEG, _EB), _f32)]
        + [pltpu.VMEM((_EB, _WH), _f32)] * _NB
        + [pltpu.VMEM_SHARED((_N_PAD, _WH), _f32)] * 2
        + [pltpu.SemaphoreType.DMA] * (2 * _NB)
    ),
)


def _dinv_body(parts_ref, dinv_ref, dinv2_ref):
    deg = 1.0 + parts_ref[0] + parts_ref[1]
    dv = jnp.where(deg > 0, lax.rsqrt(deg), 0.0)
    dinv_ref[...] = dv
    dinv2_ref[...] = dv * dv


def _dinv_call(parts):
    return pl.pallas_call(
        _dinv_body,
        grid=(_N_PAD // 128,),
        in_specs=[pl.BlockSpec((_NC, 128, 1), lambda i: (0, i, 0))],
        out_specs=[pl.BlockSpec((128, 1), lambda i: (i, 0)),
                   pl.BlockSpec((128, 1), lambda i: (i, 0))],
        out_shape=[jax.ShapeDtypeStruct((_N_PAD, 1), _f32),
                   jax.ShapeDtypeStruct((_N_PAD, 1), _f32)],
    )(parts)


def _mm_body(x_ref, w_ref, o_ref):
    o_ref[...] = jnp.dot(x_ref[...], w_ref[...],
                         preferred_element_type=_f32)


def _mm_call(x, w):
    wo = w.shape[1]
    return pl.pallas_call(
        _mm_body,
        grid=(_N_PAD // 128,),
        in_specs=[pl.BlockSpec((128, _D), lambda i: (i, 0)),
                  pl.BlockSpec((_D, wo), lambda i: (0, 0))],
        out_specs=pl.BlockSpec((128, wo), lambda i: (i, 0)),
        out_shape=jax.ShapeDtypeStruct((_N_PAD, wo), _f32),
    )(x, w)


def _epi_body(p_ref, z_ref, d2_ref, b_ref, w_ref, o_ref):
    agg = jnp.concatenate(
        [p_ref[0, 0] + p_ref[0, 1], p_ref[1, 0] + p_ref[1, 1]], axis=-1)
    h = agg + d2_ref[...] * z_ref[...] + b_ref[...]
    h = jnp.maximum(h, 0.0)
    o_ref[...] = jnp.dot(h, w_ref[...], preferred_element_type=_f32)


def _epi_call(p, z, d2, b, w):
    wo = w.shape[1]
    return pl.pallas_call(
        _epi_body,
        grid=(_N_PAD // 128,),
        in_specs=[pl.BlockSpec((2, _NC, 128, _WH), lambda i: (0, 0, i, 0)),
                  pl.BlockSpec((128, _H), lambda i: (i, 0)),
                  pl.BlockSpec((128, 1), lambda i: (i, 0)),
                  pl.BlockSpec((1, _H), lambda i: (0, 0)),
                  pl.BlockSpec((_H, wo), lambda i: (0, 0))],
        out_specs=pl.BlockSpec((128, wo), lambda i: (i, 0)),
        out_shape=jax.ShapeDtypeStruct((_N_PAD, wo), _f32),
    )(p, z, d2, b, w)


def _final_body(p0_ref, p1_ref, z_ref, d2_ref, b_ref, o_ref):
    logits = p0_ref[...] + p1_ref[...] + d2_ref[...] * z_ref[...] + b_ref[...]
    col = lax.broadcasted_iota(_i32, (128, _CP), 1)
    valid = col < _C
    neg = jnp.float32(-1e30)
    m = jnp.max(jnp.where(valid, logits, neg), axis=-1, keepdims=True)
    ex = jnp.where(valid, jnp.exp(logits - m), 0.0)
    lse = jnp.log(jnp.sum(ex, axis=-1, keepdims=True))
    o_ref[...] = logits - m - lse


def _final_call(p0, p1, z, d2, b):
    return pl.pallas_call(
        _final_body,
        grid=(_N_PAD // 128,),
        in_specs=[pl.BlockSpec((128, _CP), lambda i: (i, 0)),
                  pl.BlockSpec((128, _CP), lambda i: (i, 0)),
                  pl.BlockSpec((128, _CP), lambda i: (i, 0)),
                  pl.BlockSpec((128, 1), lambda i: (i, 0)),
                  pl.BlockSpec((1, _CP), lambda i: (0, 0))],
        out_specs=pl.BlockSpec((128, _CP), lambda i: (i, 0)),
        out_shape=jax.ShapeDtypeStruct((_N_PAD, _CP), _f32),
    )(p0, p1, z, d2, b)


def kernel(x, edge_index, edge_attr, W1, b1, W2, b2, W3, b3):
    row = edge_index[0]
    col = edge_index[1]
    rpad = jnp.pad(row, (0, _E_PAD - _E))
    cpad = jnp.pad(col, (0, _E_PAD - _E))
    epad = jnp.pad(edge_attr, (0, _E_PAD - _E))
    row3 = rpad.reshape(_NW, _CH, _EB)
    col3 = cpad.reshape(_NW, _CH, _EB)
    ew3 = epad.reshape(_NW, _CH, _EB)
    rowf = rpad.reshape(_NW, _EPT)
    colf = cpad.reshape(_NW, _EPT)
    ewf = epad.reshape(_NW, _EPT)
    xp = jnp.pad(x, ((0, _N_PAD - _N), (0, 0)))
    W3p = jnp.pad(W3, ((0, 0), (0, _CP - _C)))
    b3p = jnp.pad(b3, (0, _CP - _C))

    deg_parts = _deg_call(col3, ew3)
    dinv, dinv2 = _dinv_call(deg_parts.reshape(_NC, _N_PAD, 1))
    z1 = _mm_call(xp, W1)
    norm = _norm_call(rowf, colf, ewf, dinv.reshape(_N_PAD))
    norm3 = norm.reshape(_NW, _CH, _EB)

    p1 = _agg2(z1[:, :_WH], z1[:, _WH:], row3, col3, norm3)
    z2 = _epi_call(p1, z1, dinv2, b1.reshape(1, _H), W2)
    p2 = _agg2(z2[:, :_WH], z2[:, _WH:], row3, col3, norm3)
    z3 = _epi_call(p2, z2, dinv2, b2.reshape(1, _H), W3p)
    p3 = _agg48(z3, row3, col3, norm3)
    out = _final_call(p3[0], p3[1], z3, dinv2, b3p.reshape(1, _CP))
    return out[:_N, :_C]


# speedup vs baseline: 1.9745x; 1.0432x over previous
import functools

import jax
import jax.numpy as jnp
from jax import lax
from jax.experimental import pallas as pl
from jax.experimental.pallas import tpu as pltpu
from jax.experimental.pallas import tpu_sc as plsc

_N = 10000
_E = 320000
_D = 128
_H = 128
_C = 40

_NC = 2
_NS = 16
_NW = _NC * _NS
_EB = 64
_CH = 160
_SEG = 80
_NB = 5
_EPT = _CH * _EB
_E_PAD = _NW * _EPT
_N_PAD = 10240
_RPT = _N_PAD // _NS
_CP = 48

_f32 = jnp.float32
_i32 = jnp.int32

_mesh = plsc.VectorSubcoreMesh(
    core_axis_name="c", subcore_axis_name="s",
    num_cores=_NC, num_subcores=_NS)

_sc_params = pltpu.CompilerParams(needs_layout_passes=False,
                                  use_tc_tiling_on_sc=False)


def _deg_body(col_hbm, ew_hbm, parts_hbm, cidx, ewv, zb, acc):
    c = lax.axis_index("c")
    s = lax.axis_index("s")
    w = c * _NS + s
    for t in range(_RPT // 16):
        zb[pl.ds(t * 16, 16)] = jnp.zeros((16,), _f32)
    pltpu.sync_copy(zb, acc.at[pl.ds(s * _RPT, _RPT)])
    plsc.subcore_barrier()
    pltpu.sync_copy(col_hbm.at[w], cidx)
    pltpu.sync_copy(ew_hbm.at[w], ewv)

    def j_body(j, carry):
        pltpu.sync_copy(ewv.at[j], acc.at[cidx.at[j]], add=True)
        return carry

    lax.fori_loop(0, _CH, j_body, 0)
    plsc.subcore_barrier()
    pltpu.sync_copy(acc.at[pl.ds(s * _RPT, _RPT)],
                    parts_hbm.at[c, pl.ds(s * _RPT, _RPT)])


_deg_call = pl.kernel(
    _deg_body,
    out_type=jax.ShapeDtypeStruct((_NC, _N_PAD), _f32),
    mesh=_mesh,
    compiler_params=_sc_params,
    scratch_types=[
        pltpu.VMEM((_CH, _EB), _i32),
        pltpu.VMEM((_CH, _EB), _f32),
        pltpu.VMEM((_RPT,), _f32),
        pltpu.VMEM_SHARED((_N_PAD,), _f32),
    ],
)


def _norm_body(row_hbm, col_hbm, ew_hbm, dinv_hbm, norm_hbm, dv, rv, cv, ev, nv):
    c = lax.axis_index("c")
    s = lax.axis_index("s")
    w = c * _NS + s
    pltpu.sync_copy(dinv_hbm, dv)
    pltpu.sync_copy(row_hbm.at[w], rv)
    pltpu.sync_copy(col_hbm.at[w], cv)
    pltpu.sync_copy(ew_hbm.at[w], ev)

    def t_body(t, carry):
        r16 = rv[pl.ds(t * 16, 16)]
        c16 = cv[pl.ds(t * 16, 16)]
        e16 = ev[pl.ds(t * 16, 16)]
        n16 = plsc.load_gather(dv, [r16]) * e16 * plsc.load_gather(dv, [c16])
        nv[pl.ds(t * 16, 16)] = n16
        return carry

    lax.fori_loop(0, _EPT // 16, t_body, 0)
    pltpu.sync_copy(nv, norm_hbm.at[w])


_norm_call = pl.kernel(
    _norm_body,
    out_type=jax.ShapeDtypeStruct((_NW, _EPT), _f32),
    mesh=_mesh,
    compiler_params=_sc_params,
    scratch_types=[
        pltpu.VMEM((_N_PAD,), _f32),
        pltpu.VMEM((_EPT,), _i32),
        pltpu.VMEM((_EPT,), _i32),
        pltpu.VMEM((_EPT,), _f32),
        pltpu.VMEM((_EPT,), _f32),
    ],
)


def _agg_body(Wd, spmem_z, z_hbm, row_hbm, col_hbm, norm_hbm, parts_hbm,
              *refs):
    if spmem_z:
        (ridx, cidx, nv, gb0, gb1, gb2, gb3, gb4, acc, zsh,
         gs0, gs1, gs2, gs3, gs4, ss0, ss1, ss2, ss3, ss4) = refs
    else:
        (ridx, cidx, nv, gb0, gb1, gb2, gb3, gb4, acc,
         gs0, gs1, gs2, gs3, gs4, ss0, ss1, ss2, ss3, ss4) = refs
        zsh = None
    c = lax.axis_index("c")
    s = lax.axis_index("s")
    w = c * _NS + s
    gbs = (gb0, gb1, gb2, gb3, gb4)
    gss = (gs0, gs1, gs2, gs3, gs4)
    sss = (ss0, ss1, ss2, ss3, ss4)

    if spmem_z:
        pltpu.sync_copy(z_hbm.at[pl.ds(s * _RPT, _RPT)],
                        zsh.at[pl.ds(s * _RPT, _RPT)])
    zsrc = zsh if spmem_z else z_hbm

    def zrow(r, carry):
        for v in range(Wd // 16):
            gb0[r, pl.ds(v * 16, 16)] = jnp.zeros((16,), _f32)
        return carry

    lax.fori_loop(0, _EB, zrow, 0)

    def zcopy(k, carry):
        pltpu.sync_copy(gb0, acc.at[pl.ds(s * _RPT + k * _EB, _EB)])
        return carry

    lax.fori_loop(0, _RPT // _EB, zcopy, 0)
    plsc.subcore_barrier()

    def stage(seg):
        base = seg * _SEG
        pltpu.sync_copy(row_hbm.at[w, pl.ds(base, _SEG)], ridx)
        pltpu.sync_copy(col_hbm.at[w, pl.ds(base, _SEG)], cidx)
        pltpu.sync_copy(norm_hbm.at[w, pl.ds(base, _SEG)], nv)

    def start_gather(l, b):
        pltpu.async_copy(zsrc.at[ridx.at[l]], gbs[b], gss[b])

    def wait_gather(l, b):
        pltpu.make_async_copy(zsrc.at[ridx.at[l]], gbs[b], gss[b]).wait()

    def scale(l, b):
        gb = gbs[b]

        @plsc.parallel_loop(0, _EB, 1, unroll=4)
        def _(e):
            nspl = plsc.load_gather(
                nv, [jnp.full((16,), l, _i32), jnp.full((16,), e, _i32)])
            for v in range(Wd // 16):
                sl = pl.ds(v * 16, 16)
                gb[e, sl] = gb[e, sl] * nspl

    def start_scatter(l, b):
        pltpu.async_copy(gbs[b], acc.at[cidx.at[l]], sss[b], add=True)

    def wait_scatter(l, b):
        pltpu.make_async_copy(gbs[b], acc.at[cidx.at[l]], sss[b]).wait()

    def run_seg():
        for b in range(3):
            start_gather(b, b)

        def k_body(k, carry):
            for phase in range(_NB):
                l = k * _NB + phase
                wait_gather(l, phase)
                scale(l, phase)
                start_scatter(l, phase)
                pb = (phase + 3) % _NB

                @pl.when(l >= 2)
                def _():
                    wait_scatter(l - 2, pb)

                @pl.when(l + 3 < _SEG)
                def _():
                    start_gather(l + 3, pb)

            return carry

        lax.fori_loop(0, _SEG // _NB, k_body, 0)
        wait_scatter(_SEG - 2, (_SEG - 2) % _NB)
        wait_scatter(_SEG - 1, (_SEG - 1) % _NB)

    stage(0)
    run_seg()
    stage(1)
    run_seg()
    plsc.subcore_barrier()

    def wb(k, carry):
        pltpu.sync_copy(acc.at[pl.ds(s * _RPT + k * 128, 128)],
                        parts_hbm.at[c, pl.ds(s * _RPT + k * 128, 128)])
        return carry

    lax.fori_loop(0, _RPT // 128, wb, 0)


def _make_agg(Wd, spmem_z):
    return pl.kernel(
        functools.partial(_agg_body, Wd, spmem_z),
        out_type=jax.ShapeDtypeStruct((_NC, _N_PAD, Wd), _f32),
        mesh=_mesh,
        compiler_params=_sc_params,
        scratch_types=(
            [pltpu.VMEM((_SEG, _EB), _i32),
             pltpu.VMEM((_SEG, _EB), _i32),
             pltpu.VMEM((_SEG, _EB), _f32)]
            + [pltpu.VMEM((_EB, Wd), _f32)] * _NB
            + [pltpu.VMEM_SHARED((_N_PAD, Wd), _f32)]
            + ([pltpu.VMEM_SHARED((_N_PAD, Wd), _f32)] if spmem_z else [])
            + [pltpu.SemaphoreType.DMA] * (2 * _NB)
        ),
    )


_agg48 = _make_agg(_CP, True)
_WH = 64


def _agg2_body(zlo_hbm, zhi_hbm, row_hbm, col_hbm, norm_hbm, parts_hbm,
               ridx, cidx, nv,
               gb0, gb1, gb2, gb3, gb4, acc, zsh,
               gs0, gs1, gs2, gs3, gs4,
               ss0, ss1, ss2, ss3, ss4):
    c = lax.axis_index("c")
    s = lax.axis_index("s")
    w = c * _NS + s
    gbs = (gb0, gb1, gb2, gb3, gb4)
    gss = (gs0, gs1, gs2, gs3, gs4)
    sss = (ss0, ss1, ss2, ss3, ss4)
    zhalves = (zlo_hbm, zhi_hbm)

    def stage(seg):
        base = seg * _SEG
        pltpu.sync_copy(row_hbm.at[w, pl.ds(base, _SEG)], ridx)
        pltpu.sync_copy(col_hbm.at[w, pl.ds(base, _SEG)], cidx)
        pltpu.sync_copy(norm_hbm.at[w, pl.ds(base, _SEG)], nv)

    def start_gather(l, b):
        pltpu.async_copy(zsh.at[ridx.at[l]], gbs[b], gss[b])

    def wait_gather(l, b):
        pltpu.make_async_copy(zsh.at[ridx.at[l]], gbs[b], gss[b]).wait()

    def scale(l, b):
        gb = gbs[b]

        @plsc.parallel_loop(0, _EB, 1, unroll=4)
        def _(e):
            nspl = plsc.load_gather(
                nv, [jnp.full((16,), l, _i32), jnp.full((16,), e, _i32)])
            for v in range(_WH // 16):
                sl = pl.ds(v * 16, 16)
                gb[e, sl] = gb[e, sl] * nspl

    def start_scatter(l, b):
        pltpu.async_copy(gbs[b], acc.at[cidx.at[l]], sss[b], add=True)

    def wait_scatter(l, b):
        pltpu.make_async_copy(gbs[b], acc.at[cidx.at[l]], sss[b]).wait()

    def run_seg():
        for b in range(3):
            start_gather(b, b)

        def k_body(k, carry):
            for phase in range(_NB):
                l = k * _NB + phase
                wait_gather(l, phase)
                scale(l, phase)
                start_scatter(l, phase)
                pb = (phase + 3) % _NB

                @pl.when(l >= 2)
                def _():
                    wait_scatter(l - 2, pb)

                @pl.when(l + 3 < _SEG)
                def _():
                    start_gather(l + 3, pb)

            return carry

        lax.fori_loop(0, _SEG // _NB, k_body, 0)
        wait_scatter(_SEG - 2, (_SEG - 2) % _NB)
        wait_scatter(_SEG - 1, (_SEG - 1) % _NB)

    for half in range(2):
        pltpu.sync_copy(zhalves[half].at[pl.ds(s * _RPT, _RPT)],
                        zsh.at[pl.ds(s * _RPT, _RPT)])

        def zrow(r, carry):
            for v in range(_WH // 16):
                gb0[r, pl.ds(v * 16, 16)] = jnp.zeros((16,), _f32)
            return carry

        lax.fori_loop(0, _EB, zrow, 0)

        def zcopy(k, carry):
            pltpu.sync_copy(gb0, acc.at[pl.ds(s * _RPT + k * _EB, _EB)])
            return carry

        lax.fori_loop(0, _RPT // _EB, zcopy, 0)
        plsc.subcore_barrier()

        stage(0)
        run_seg()
        stage(1)
        run_seg()
        plsc.subcore_barrier()

        def wb(k, carry):
            pltpu.sync_copy(
                acc.at[pl.ds(s * _RPT + k * 128, 128)],
                parts_hbm.at[half, c, pl.ds(s * _RPT + k * 128, 128)])
            return carry

        lax.fori_loop(0, _RPT // 128, wb, 0)


_agg2 = pl.kernel(
    _agg2_body,
    out_type=jax.ShapeDtypeStruct((2, _NC, _N_PAD, _WH), _f32),
    mesh=_mesh,
    compiler_params=_sc_params,
    scratch_types=(
        [pltpu.VMEM((_SEG, _EB), _i32),
         pltpu.VMEM((_SEG, _EB), _i32),
         pltpu.VMEM((_SEG, _EB), _f32)]
        + [pltpu.VMEM((_EB, _WH), _f32)] * _NB
        + [pltpu.VMEM_SHARED((_N_PAD, _WH), _f32)] * 2
        + [pltpu.SemaphoreType.DMA] * (2 * _NB)
    ),
)


def _dinv_mm_body(parts_ref, x_ref, w_ref, z_ref, dinv_ref, dinv2_ref):
    deg = 1.0 + parts_ref[0] + parts_ref[1]
    dv = jnp.where(deg > 0, lax.rsqrt(deg), 0.0)
    dinv_ref[...] = dv
    dinv2_ref[...] = dv * dv
    z_ref[...] = jnp.dot(x_ref[...], w_ref[...],
                         preferred_element_type=_f32)


def _dinv_mm_call(parts, x, w):
    wo = w.shape[1]
    return pl.pallas_call(
        _dinv_mm_body,
        grid=(_N_PAD // 128,),
        in_specs=[pl.BlockSpec((_NC, 128, 1), lambda i: (0, i, 0)),
                  pl.BlockSpec((128, _D), lambda i: (i, 0)),
                  pl.BlockSpec((_D, wo), lambda i: (0, 0))],
        out_specs=[pl.BlockSpec((128, wo), lambda i: (i, 0)),
                   pl.BlockSpec((128, 1), lambda i: (i, 0)),
                   pl.BlockSpec((128, 1), lambda i: (i, 0))],
        out_shape=[jax.ShapeDtypeStruct((_N_PAD, wo), _f32),
                   jax.ShapeDtypeStruct((_N_PAD, 1), _f32),
                   jax.ShapeDtypeStruct((_N_PAD, 1), _f32)],
    )(parts, x, w)


def _epi_body(p_ref, z_ref, d2_ref, b_ref, w_ref, o_ref):
    agg = jnp.concatenate(
        [p_ref[0, 0] + p_ref[0, 1], p_ref[1, 0] + p_ref[1, 1]], axis=-1)
    h = agg + d2_ref[...] * z_ref[...] + b_ref[...]
    h = jnp.maximum(h, 0.0)
    o_ref[...] = jnp.dot(h, w_ref[...], preferred_element_type=_f32)


def _epi_call(p, z, d2, b, w):
    wo = w.shape[1]
    return pl.pallas_call(
        _epi_body,
        grid=(_N_PAD // 128,),
        in_specs=[pl.BlockSpec((2, _NC, 128, _WH), lambda i: (0, 0, i, 0)),
                  pl.BlockSpec((128, _H), lambda i: (i, 0)),
                  pl.BlockSpec((128, 1), lambda i: (i, 0)),
                  pl.BlockSpec((1, _H), lambda i: (0, 0)),
                  pl.BlockSpec((_H, wo), lambda i: (0, 0))],
        out_specs=pl.BlockSpec((128, wo), lambda i: (i, 0)),
        out_shape=jax.ShapeDtypeStruct((_N_PAD, wo), _f32),
    )(p, z, d2, b, w)


def _final_body(p0_ref, p1_ref, z_ref, d2_ref, b_ref, o_ref):
    logits = p0_ref[...] + p1_ref[...] + d2_ref[...] * z_ref[...] + b_ref[...]
    col = lax.broadcasted_iota(_i32, (128, _CP), 1)
    valid = col < _C
    neg = jnp.float32(-1e30)
    m = jnp.max(jnp.where(valid, logits, neg), axis=-1, keepdims=True)
    ex = jnp.where(valid, jnp.exp(logits - m), 0.0)
    lse = jnp.log(jnp.sum(ex, axis=-1, keepdims=True))
    o_ref[...] = logits - m - lse


def _final_call(p0, p1, z, d2, b):
    return pl.pallas_call(
        _final_body,
        grid=(_N_PAD // 128,),
        in_specs=[pl.BlockSpec((128, _CP), lambda i: (i, 0)),
                  pl.BlockSpec((128, _CP), lambda i: (i, 0)),
                  pl.BlockSpec((128, _CP), lambda i: (i, 0)),
                  pl.BlockSpec((128, 1), lambda i: (i, 0)),
                  pl.BlockSpec((1, _CP), lambda i: (0, 0))],
        out_specs=pl.BlockSpec((128, _CP), lambda i: (i, 0)),
        out_shape=jax.ShapeDtypeStruct((_N_PAD, _CP), _f32),
    )(p0, p1, z, d2, b)


def kernel(x, edge_index, edge_attr, W1, b1, W2, b2, W3, b3):
    row = edge_index[0]
    col = edge_index[1]
    rpad = jnp.pad(row, (0, _E_PAD - _E))
    cpad = jnp.pad(col, (0, _E_PAD - _E))
    epad = jnp.pad(edge_attr, (0, _E_PAD - _E))
    row3 = rpad.reshape(_NW, _CH, _EB)
    col3 = cpad.reshape(_NW, _CH, _EB)
    ew3 = epad.reshape(_NW, _CH, _EB)
    rowf = rpad.reshape(_NW, _EPT)
    colf = cpad.reshape(_NW, _EPT)
    ewf = epad.reshape(_NW, _EPT)
    xp = jnp.pad(x, ((0, _N_PAD - _N), (0, 0)))
    W3p = jnp.pad(W3, ((0, 0), (0, _CP - _C)))
    b3p = jnp.pad(b3, (0, _CP - _C))

    deg_parts = _deg_call(col3, ew3)
    z1, dinv, dinv2 = _dinv_mm_call(
        deg_parts.reshape(_NC, _N_PAD, 1), xp, W1)
    norm = _norm_call(rowf, colf, ewf, dinv.reshape(_N_PAD))
    norm3 = norm.reshape(_NW, _CH, _EB)

    p1 = _agg2(z1[:, :_WH], z1[:, _WH:], row3, col3, norm3)
    z2 = _epi_call(p1, z1, dinv2, b1.reshape(1, _H), W2)
    p2 = _agg2(z2[:, :_WH], z2[:, _WH:], row3, col3, norm3)
    z3 = _epi_call(p2, z2, dinv2, b2.reshape(1, _H), W3p)
    p3 = _agg48(z3, row3, col3, norm3)
    out = _final_call(p3[0], p3[1], z3, dinv2, b3p.reshape(1, _CP))
    return out[:_N, :_C]


# speedup vs baseline: 1.9756x; 1.0005x over previous
import functools

import jax
import jax.numpy as jnp
from jax import lax
from jax.experimental import pallas as pl
from jax.experimental.pallas import tpu as pltpu
from jax.experimental.pallas import tpu_sc as plsc

_N = 10000
_E = 320000
_D = 128
_H = 128
_C = 40

_NC = 2
_NS = 16
_NW = _NC * _NS
_EB = 64
_CH = 160
_SEG = 80
_NB = 5
_EPT = _CH * _EB
_E_PAD = _NW * _EPT
_N_PAD = 10240
_RPT = _N_PAD // _NS
_CP = 48

_f32 = jnp.float32
_i32 = jnp.int32

_mesh = plsc.VectorSubcoreMesh(
    core_axis_name="c", subcore_axis_name="s",
    num_cores=_NC, num_subcores=_NS)

_sc_params = pltpu.CompilerParams(needs_layout_passes=False,
                                  use_tc_tiling_on_sc=False)


def _deg_body(col_hbm, ew_hbm, parts_hbm, cidx, ewv, zb, acc):
    c = lax.axis_index("c")
    s = lax.axis_index("s")
    w = c * _NS + s
    for t in range(_RPT // 16):
        zb[pl.ds(t * 16, 16)] = jnp.zeros((16,), _f32)
    pltpu.sync_copy(zb, acc.at[pl.ds(s * _RPT, _RPT)])
    plsc.subcore_barrier()
    pltpu.sync_copy(col_hbm.at[w], cidx)
    pltpu.sync_copy(ew_hbm.at[w], ewv)

    def j_body(j, carry):
        pltpu.sync_copy(ewv.at[j], acc.at[cidx.at[j]], add=True)
        return carry

    lax.fori_loop(0, _CH, j_body, 0)
    plsc.subcore_barrier()
    pltpu.sync_copy(acc.at[pl.ds(s * _RPT, _RPT)],
                    parts_hbm.at[c, pl.ds(s * _RPT, _RPT)])


_deg_call = pl.kernel(
    _deg_body,
    out_type=jax.ShapeDtypeStruct((_NC, _N_PAD), _f32),
    mesh=_mesh,
    compiler_params=_sc_params,
    scratch_types=[
        pltpu.VMEM((_CH, _EB), _i32),
        pltpu.VMEM((_CH, _EB), _f32),
        pltpu.VMEM((_RPT,), _f32),
        pltpu.VMEM_SHARED((_N_PAD,), _f32),
    ],
)


def _norm_body(row_hbm, col_hbm, ew_hbm, dinv_hbm, norm_hbm, dv, rv, cv, ev, nv):
    c = lax.axis_index("c")
    s = lax.axis_index("s")
    w = c * _NS + s
    pltpu.sync_copy(dinv_hbm, dv)
    pltpu.sync_copy(row_hbm.at[w], rv)
    pltpu.sync_copy(col_hbm.at[w], cv)
    pltpu.sync_copy(ew_hbm.at[w], ev)

    def t_body(t, carry):
        r16 = rv[pl.ds(t * 16, 16)]
        c16 = cv[pl.ds(t * 16, 16)]
        e16 = ev[pl.ds(t * 16, 16)]
        n16 = plsc.load_gather(dv, [r16]) * e16 * plsc.load_gather(dv, [c16])
        nv[pl.ds(t * 16, 16)] = n16
        return carry

    lax.fori_loop(0, _EPT // 16, t_body, 0)
    pltpu.sync_copy(nv, norm_hbm.at[w])


_norm_call = pl.kernel(
    _norm_body,
    out_type=jax.ShapeDtypeStruct((_NW, _EPT), _f32),
    mesh=_mesh,
    compiler_params=_sc_params,
    scratch_types=[
        pltpu.VMEM((_N_PAD,), _f32),
        pltpu.VMEM((_EPT,), _i32),
        pltpu.VMEM((_EPT,), _i32),
        pltpu.VMEM((_EPT,), _f32),
        pltpu.VMEM((_EPT,), _f32),
    ],
)


def _agg_body(Wd, z_hbm, row_hbm, col_hbm, norm_hbm, parts_hbm,
              ridx, cidx, nv, gb0, gb1, gb2, gb3, gb4, acc, zsh,
              gs0, gs1, gs2, gs3, gs4, ss0, ss1, ss2, ss3, ss4):
    c = lax.axis_index("c")
    s = lax.axis_index("s")
    w = c * _NS + s
    gbs = (gb0, gb1, gb2, gb3, gb4)
    gss = (gs0, gs1, gs2, gs3, gs4)
    sss = (ss0, ss1, ss2, ss3, ss4)

    pltpu.sync_copy(z_hbm.at[pl.ds(s * _RPT, _RPT)],
                    zsh.at[pl.ds(s * _RPT, _RPT)])
    zsrc = zsh

    def zrow(r, carry):
        for v in range(Wd // 16):
            gb0[r, pl.ds(v * 16, 16)] = jnp.zeros((16,), _f32)
        return carry

    lax.fori_loop(0, _EB, zrow, 0)

    def zcopy(k, carry):
        pltpu.sync_copy(gb0, acc.at[pl.ds(s * _RPT + k * _EB, _EB)])
        return carry

    lax.fori_loop(0, _RPT // _EB, zcopy, 0)
    plsc.subcore_barrier()

    def stage(seg):
        base = seg * _SEG
        pltpu.sync_copy(row_hbm.at[w, pl.ds(base, _SEG)], ridx)
        pltpu.sync_copy(col_hbm.at[w, pl.ds(base, _SEG)], cidx)
        pltpu.sync_copy(norm_hbm.at[w, pl.ds(base, _SEG)], nv)

    def start_gather(l, b):
        pltpu.async_copy(zsrc.at[ridx.at[l]], gbs[b], gss[b])

    def wait_gather(l, b):
        pltpu.make_async_copy(zsrc.at[ridx.at[l]], gbs[b], gss[b]).wait()

    def scale(l, b):
        gb = gbs[b]

        @plsc.parallel_loop(0, _EB, 1, unroll=4)
        def _(e):
            nspl = plsc.load_gather(
                nv, [jnp.full((16,), l, _i32), jnp.full((16,), e, _i32)])
            for v in range(Wd // 16):
                sl = pl.ds(v * 16, 16)
                gb[e, sl] = gb[e, sl] * nspl

    def start_scatter(l, b):
        pltpu.async_copy(gbs[b], acc.at[cidx.at[l]], sss[b], add=True)

    def wait_scatter(l, b):
        pltpu.make_async_copy(gbs[b], acc.at[cidx.at[l]], sss[b]).wait()

    def run_seg():
        for b in range(3):
            start_gather(b, b)

        def k_body(k, carry):
            for phase in range(_NB):
                l = k * _NB + phase
                wait_gather(l, phase)
                scale(l, phase)
                start_scatter(l, phase)
                pb = (phase + 3) % _NB

                @pl.when(l >= 2)
                def _():
                    wait_scatter(l - 2, pb)

                @pl.when(l + 3 < _SEG)
                def _():
                    start_gather(l + 3, pb)

            return carry

        lax.fori_loop(0, _SEG // _NB, k_body, 0)
        wait_scatter(_SEG - 2, (_SEG - 2) % _NB)
        wait_scatter(_SEG - 1, (_SEG - 1) % _NB)

    stage(0)
    run_seg()
    stage(1)
    run_seg()
    plsc.subcore_barrier()

    def wb(k, carry):
        pltpu.sync_copy(acc.at[pl.ds(s * _RPT + k * 128, 128)],
                        parts_hbm.at[c, pl.ds(s * _RPT + k * 128, 128)])
        return carry

    lax.fori_loop(0, _RPT // 128, wb, 0)


def _make_agg(Wd):
    return pl.kernel(
        functools.partial(_agg_body, Wd),
        out_type=jax.ShapeDtypeStruct((_NC, _N_PAD, Wd), _f32),
        mesh=_mesh,
        compiler_params=_sc_params,
        scratch_types=(
            [pltpu.VMEM((_SEG, _EB), _i32),
             pltpu.VMEM((_SEG, _EB), _i32),
             pltpu.VMEM((_SEG, _EB), _f32)]
            + [pltpu.VMEM((_EB, Wd), _f32)] * _NB
            + [pltpu.VMEM_SHARED((_N_PAD, Wd), _f32)] * 2
            + [pltpu.SemaphoreType.DMA] * (2 * _NB)
        ),
    )


_agg48 = _make_agg(_CP)
_WH = 64


def _agg2_body(zlo_hbm, zhi_hbm, row_hbm, col_hbm, norm_hbm, parts_hbm,
               ridx, cidx, nv,
               gb0, gb1, gb2, gb3, gb4, acc, zsh,
               gs0, gs1, gs2, gs3, gs4,
               ss0, ss1, ss2, ss3, ss4):
    c = lax.axis_index("c")
    s = lax.axis_index("s")
    w = c * _NS + s
    gbs = (gb0, gb1, gb2, gb3, gb4)
    gss = (gs0, gs1, gs2, gs3, gs4)
    sss = (ss0, ss1, ss2, ss3, ss4)
    zhalves = (zlo_hbm, zhi_hbm)

    def stage(seg):
        base = seg * _SEG
        pltpu.sync_copy(row_hbm.at[w, pl.ds(base, _SEG)], ridx)
        pltpu.sync_copy(col_hbm.at[w, pl.ds(base, _SEG)], cidx)
        pltpu.sync_copy(norm_hbm.at[w, pl.ds(base, _SEG)], nv)

    def start_gather(l, b):
        pltpu.async_copy(zsh.at[ridx.at[l]], gbs[b], gss[b])

    def wait_gather(l, b):
        pltpu.make_async_copy(zsh.at[ridx.at[l]], gbs[b], gss[b]).wait()

    def scale(l, b):
        gb = gbs[b]

        @plsc.parallel_loop(0, _EB, 1, unroll=4)
        def _(e):
            nspl = plsc.load_gather(
                nv, [jnp.full((16,), l, _i32), jnp.full((16,), e, _i32)])
            for v in range(_WH // 16):
                sl = pl.ds(v * 16, 16)
                gb[e, sl] = gb[e, sl] * nspl

    def start_scatter(l, b):
        pltpu.async_copy(gbs[b], acc.at[cidx.at[l]], sss[b], add=True)

    def wait_scatter(l, b):
        pltpu.make_async_copy(gbs[b], acc.at[cidx.at[l]], sss[b]).wait()

    def run_seg():
        for b in range(3):
            start_gather(b, b)

        def k_body(k, carry):
            for phase in range(_NB):
                l = k * _NB + phase
                wait_gather(l, phase)
                scale(l, phase)
                start_scatter(l, phase)
                pb = (phase + 3) % _NB

                @pl.when(l >= 2)
                def _():
                    wait_scatter(l - 2, pb)

                @pl.when(l + 3 < _SEG)
                def _():
                    start_gather(l + 3, pb)

            return carry

        lax.fori_loop(0, _SEG // _NB, k_body, 0)
        wait_scatter(_SEG - 2, (_SEG - 2) % _NB)
        wait_scatter(_SEG - 1, (_SEG - 1) % _NB)

    for half in range(2):
        pltpu.sync_copy(zhalves[half].at[pl.ds(s * _RPT, _RPT)],
                        zsh.at[pl.ds(s * _RPT, _RPT)])

        def zrow(r, carry):
            for v in range(_WH // 16):
                gb0[r, pl.ds(v * 16, 16)] = jnp.zeros((16,), _f32)
            return carry

        lax.fori_loop(0, _EB, zrow, 0)

        def zcopy(k, carry):
            pltpu.sync_copy(gb0, acc.at[pl.ds(s * _RPT + k * _EB, _EB)])
            return carry

        lax.fori_loop(0, _RPT // _EB, zcopy, 0)
        plsc.subcore_barrier()

        stage(0)
        run_seg()
        stage(1)
        run_seg()
        plsc.subcore_barrier()

        def wb(k, carry):
            pltpu.sync_copy(
                acc.at[pl.ds(s * _RPT + k * 128, 128)],
                parts_hbm.at[half, c, pl.ds(s * _RPT + k * 128, 128)])
            return carry

        lax.fori_loop(0, _RPT // 128, wb, 0)


_agg2 = pl.kernel(
    _agg2_body,
    out_type=jax.ShapeDtypeStruct((2, _NC, _N_PAD, _WH), _f32),
    mesh=_mesh,
    compiler_params=_sc_params,
    scratch_types=(
        [pltpu.VMEM((_SEG, _EB), _i32),
         pltpu.VMEM((_SEG, _EB), _i32),
         pltpu.VMEM((_SEG, _EB), _f32)]
        + [pltpu.VMEM((_EB, _WH), _f32)] * _NB
        + [pltpu.VMEM_SHARED((_N_PAD, _WH), _f32)] * 2
        + [pltpu.SemaphoreType.DMA] * (2 * _NB)
    ),
)


def _dinv_mm_body(parts_ref, x_ref, w_ref, z_ref, dinv_ref, dinv2_ref):
    deg = 1.0 + parts_ref[0] + parts_ref[1]
    dv = jnp.where(deg > 0, lax.rsqrt(deg), 0.0)
    dinv_ref[...] = dv
    dinv2_ref[...] = dv * dv
    z_ref[...] = jnp.dot(x_ref[...], w_ref[...],
                         preferred_element_type=_f32)


def _dinv_mm_call(parts, x, w):
    wo = w.shape[1]
    return pl.pallas_call(
        _dinv_mm_body,
        grid=(_N_PAD // 128,),
        in_specs=[pl.BlockSpec((_NC, 128, 1), lambda i: (0, i, 0)),
                  pl.BlockSpec((128, _D), lambda i: (i, 0)),
                  pl.BlockSpec((_D, wo), lambda i: (0, 0))],
        out_specs=[pl.BlockSpec((128, wo), lambda i: (i, 0)),
                   pl.BlockSpec((128, 1), lambda i: (i, 0)),
                   pl.BlockSpec((128, 1), lambda i: (i, 0))],
        out_shape=[jax.ShapeDtypeStruct((_N_PAD, wo), _f32),
                   jax.ShapeDtypeStruct((_N_PAD, 1), _f32),
                   jax.ShapeDtypeStruct((_N_PAD, 1), _f32)],
    )(parts, x, w)


def _epi_body(p_ref, z_ref, d2_ref, b_ref, w_ref, o_ref):
    agg = jnp.concatenate(
        [p_ref[0, 0] + p_ref[0, 1], p_ref[1, 0] + p_ref[1, 1]], axis=-1)
    h = agg + d2_ref[...] * z_ref[...] + b_ref[...]
    h = jnp.maximum(h, 0.0)
    o_ref[...] = jnp.dot(h, w_ref[...], preferred_element_type=_f32)


def _epi_call(p, z, d2, b, w):
    wo = w.shape[1]
    return pl.pallas_call(
        _epi_body,
        grid=(_N_PAD // 128,),
        in_specs=[pl.BlockSpec((2, _NC, 128, _WH), lambda i: (0, 0, i, 0)),
                  pl.BlockSpec((128, _H), lambda i: (i, 0)),
                  pl.BlockSpec((128, 1), lambda i: (i, 0)),
                  pl.BlockSpec((1, _H), lambda i: (0, 0)),
                  pl.BlockSpec((_H, wo), lambda i: (0, 0))],
        out_specs=pl.BlockSpec((128, wo), lambda i: (i, 0)),
        out_shape=jax.ShapeDtypeStruct((_N_PAD, wo), _f32),
    )(p, z, d2, b, w)


def _final_body(p0_ref, p1_ref, z_ref, d2_ref, b_ref, o_ref):
    logits = p0_ref[...] + p1_ref[...] + d2_ref[...] * z_ref[...] + b_ref[...]
    col = lax.broadcasted_iota(_i32, (128, _CP), 1)
    valid = col < _C
    neg = jnp.float32(-1e30)
    m = jnp.max(jnp.where(valid, logits, neg), axis=-1, keepdims=True)
    ex = jnp.where(valid, jnp.exp(logits - m), 0.0)
    lse = jnp.log(jnp.sum(ex, axis=-1, keepdims=True))
    o_ref[...] = logits - m - lse


def _final_call(p0, p1, z, d2, b):
    return pl.pallas_call(
        _final_body,
        grid=(_N_PAD // 128,),
        in_specs=[pl.BlockSpec((128, _CP), lambda i: (i, 0)),
                  pl.BlockSpec((128, _CP), lambda i: (i, 0)),
                  pl.BlockSpec((128, _CP), lambda i: (i, 0)),
                  pl.BlockSpec((128, 1), lambda i: (i, 0)),
                  pl.BlockSpec((1, _CP), lambda i: (0, 0))],
        out_specs=pl.BlockSpec((128, _CP), lambda i: (i, 0)),
        out_shape=jax.ShapeDtypeStruct((_N_PAD, _CP), _f32),
    )(p0, p1, z, d2, b)


def kernel(x, edge_index, edge_attr, W1, b1, W2, b2, W3, b3):
    row = edge_index[0]
    col = edge_index[1]
    rpad = jnp.pad(row, (0, _E_PAD - _E))
    cpad = jnp.pad(col, (0, _E_PAD - _E))
    epad = jnp.pad(edge_attr, (0, _E_PAD - _E))
    row3 = rpad.reshape(_NW, _CH, _EB)
    col3 = cpad.reshape(_NW, _CH, _EB)
    ew3 = epad.reshape(_NW, _CH, _EB)
    rowf = rpad.reshape(_NW, _EPT)
    colf = cpad.reshape(_NW, _EPT)
    ewf = epad.reshape(_NW, _EPT)
    xp = jnp.pad(x, ((0, _N_PAD - _N), (0, 0)))
    W3p = jnp.pad(W3, ((0, 0), (0, _CP - _C)))
    b3p = jnp.pad(b3, (0, _CP - _C))

    deg_parts = _deg_call(col3, ew3)
    z1, dinv, dinv2 = _dinv_mm_call(
        deg_parts.reshape(_NC, _N_PAD, 1), xp, W1)
    norm = _norm_call(rowf, colf, ewf, dinv.reshape(_N_PAD))
    norm3 = norm.reshape(_NW, _CH, _EB)

    p1 = _agg2(z1[:, :_WH], z1[:, _WH:], row3, col3, norm3)
    z2 = _epi_call(p1, z1, dinv2, b1.reshape(1, _H), W2)
    p2 = _agg2(z2[:, :_WH], z2[:, _WH:], row3, col3, norm3)
    z3 = _epi_call(p2, z2, dinv2, b2.reshape(1, _H), W3p)
    p3 = _agg48(z3, row3, col3, norm3)
    out = _final_call(p3[0], p3[1], z3, dinv2, b3p.reshape(1, _CP))
    return out[:_N, :_C]
